# Initial kernel scaffold; baseline (speedup 1.0000x reference)
import jax
import jax.numpy as jnp
from jax.experimental import pallas as pl


def kernel(reg_info, inputs, edge_index, W0, a_src0, a_dst0, W1, a_src1, a_dst1):
    raise NotImplementedError("write your pallas kernel here")


# speedup vs baseline: 6.9516x; 6.9516x over previous
import functools

import jax
import jax.numpy as jnp
from jax import lax
from jax.experimental import pallas as pl
from jax.experimental.pallas import tpu as pltpu
from jax.experimental.pallas import tpu_sc as plsc

N = 10000
E = 160000
D = 128
H = 8
HP = 16
NC = 2
NS = 16
CH = 200
NBLK = N // CH
NBI = (NBLK + NS - 1) // NS
RB = 1000

_MESH = dict(core_axis_name="c", subcore_axis_name="s", num_cores=NC,
             num_subcores=NS)


def _owned_blocks(s, body):
    def it(i, carry):
        b = i * NS + s

        @pl.when(b < NBLK)
        def _():
            body(b)

        return carry

    lax.fori_loop(0, NBI, it, 0)


def _dense0_body(x_ref, w_ref, apad_ref, h0_ref, asad_ref):
    xb = x_ref[...]
    hs = []
    for h in range(H):
        ph = jnp.dot(xb, w_ref[:, h * D:(h + 1) * D],
                     preferred_element_type=jnp.float32)
        h0_ref[h] = ph
        hs.append(ph)
    hb = jnp.concatenate(hs, axis=1)
    asad_ref[...] = jnp.dot(hb, apad_ref[...], preferred_element_type=jnp.float32)


def _dense0(x, W0, Apad):
    return pl.pallas_call(
        _dense0_body,
        grid=(N // RB,),
        in_specs=[
            pl.BlockSpec((RB, D), lambda i: (i, 0)),
            pl.BlockSpec((D, H * D), lambda i: (0, 0)),
            pl.BlockSpec((H * D, D), lambda i: (0, 0)),
        ],
        out_specs=[
            pl.BlockSpec((H, RB, D), lambda i: (0, i, 0)),
            pl.BlockSpec((RB, D), lambda i: (i, 0)),
        ],
        out_shape=[
            jax.ShapeDtypeStruct((H, N, D), jnp.float32),
            jax.ShapeDtypeStruct((N, D), jnp.float32),
        ],
    )(x, W0, Apad)


def _dense1_body(o_ref, dp0_ref, dp1_ref, w1_ref, a1pad_ref,
                 h1_ref, asad_ref):
    acc = jnp.zeros((RB, D), jnp.float32)
    for h in range(H):
        den = dp0_ref[:, h:h + 1] + dp1_ref[:, h:h + 1] + 1e-16
        xh = o_ref[h] / den
        xh = jnp.where(xh > 0, xh, jnp.exp(xh) - 1.0)
        acc = acc + jnp.dot(xh, w1_ref[h * D:(h + 1) * D, :],
                            preferred_element_type=jnp.float32)
    h1_ref[...] = acc
    asad_ref[...] = jnp.dot(acc, a1pad_ref[...], preferred_element_type=jnp.float32)


def _dense1(out0, dp0, dp1, W1, A1pad):
    return pl.pallas_call(
        _dense1_body,
        grid=(N // RB,),
        in_specs=[
            pl.BlockSpec((H, RB, D), lambda i: (0, i, 0)),
            pl.BlockSpec((RB, D), lambda i: (i, 0)),
            pl.BlockSpec((RB, D), lambda i: (i, 0)),
            pl.BlockSpec((H * D, D), lambda i: (0, 0)),
            pl.BlockSpec((D, D), lambda i: (0, 0)),
        ],
        out_specs=[
            pl.BlockSpec((RB, D), lambda i: (i, 0)),
            pl.BlockSpec((RB, D), lambda i: (i, 0)),
        ],
        out_shape=[
            jax.ShapeDtypeStruct((N, D), jnp.float32),
            jax.ShapeDtypeStruct((N, D), jnp.float32),
        ],
    )(out0, dp0, dp1, W1, A1pad)


def _sc_logits(asad, src, dst):
    B = 40
    EPT = E // (NC * NS)
    NB = EPT // B

    @functools.partial(
        pl.kernel,
        out_type=(
            jax.ShapeDtypeStruct((E, HP), jnp.float32),
            jax.ShapeDtypeStruct((N, D), jnp.float32),
            jax.ShapeDtypeStruct((N, D), jnp.float32),
        ),
        mesh=plsc.VectorSubcoreMesh(**_MESH),
        scratch_types=(
            pltpu.VMEM((B,), jnp.int32),
            pltpu.VMEM((B,), jnp.int32),
            pltpu.VMEM((B, D), jnp.float32),
            pltpu.VMEM((B, D), jnp.float32),
            pltpu.VMEM((B, D), jnp.float32),
            pltpu.VMEM((B, HP), jnp.float32),
            pltpu.VMEM((CH, D), jnp.float32),
            pltpu.VMEM_SHARED((N, D), jnp.float32),
        ),
    )
    def k(asad_hbm, src_hbm, dst_hbm, ex_hbm, dp0_hbm, dp1_hbm,
          srcb, dstb, gA, gB, exb, exs, zb, acc):
        c = lax.axis_index("c")
        s = lax.axis_index("s")
        zero = jnp.zeros((16,), jnp.float32)

        def zrow(i, carry):
            for q in range(D // 16):
                zb[i, pl.ds(q * 16, 16)] = zero
            return carry

        lax.fori_loop(0, CH, zrow, 0)
        _owned_blocks(s, lambda b: pltpu.sync_copy(zb, acc.at[pl.ds(b * CH, CH)]))

        def zrow2(i, carry):
            for q in range(D // 16):
                zb_ = exb
                zb_[i, pl.ds(q * 16, 16)] = zero
            return carry

        lax.fori_loop(0, B, zrow2, 0)
        plsc.subcore_barrier()

        base = c * (E // NC) + s * EPT

        def batch(i, carry):
            eb = base + i * B
            pltpu.sync_copy(src_hbm.at[pl.ds(eb, B)], srcb)
            pltpu.sync_copy(dst_hbm.at[pl.ds(eb, B)], dstb)
            pltpu.sync_copy(asad_hbm.at[srcb], gA)
            pltpu.sync_copy(asad_hbm.at[dstb], gB)
            for r in range(B):
                e = gA[r, pl.ds(0, 16)] + gB[r, pl.ds(16, 16)]
                e = jnp.where(e > 0.0, e, 0.2 * e)
                ex = jnp.exp(e)
                exb[r, pl.ds(0, 16)] = ex
                exs[r, :] = ex
            pltpu.sync_copy(exs, ex_hbm.at[pl.ds(eb, B)])
            pltpu.sync_copy(exb, acc.at[dstb], add=True)
            return carry

        lax.fori_loop(0, NB, batch, 0)
        plsc.subcore_barrier()

        def wb(b):
            r0 = b * CH

            @pl.when(c == 0)
            def _():
                pltpu.sync_copy(acc.at[pl.ds(r0, CH)], dp0_hbm.at[pl.ds(r0, CH)])

            @pl.when(c == 1)
            def _():
                pltpu.sync_copy(acc.at[pl.ds(r0, CH)], dp1_hbm.at[pl.ds(r0, CH)])

        _owned_blocks(s, wb)

    return k(asad, src, dst)


def _recip_body(dp0_ref, dp1_ref, o_ref):
    o_ref[...] = 1.0 / (dp0_ref[...] + dp1_ref[...] + 1e-16)


def _recip(dp0, dp1):
    return pl.pallas_call(
        _recip_body,
        grid=(N // RB,),
        in_specs=[
            pl.BlockSpec((RB, D), lambda i: (i, 0)),
            pl.BlockSpec((RB, D), lambda i: (i, 0)),
        ],
        out_specs=pl.BlockSpec((RB, D), lambda i: (i, 0)),
        out_shape=jax.ShapeDtypeStruct((N, D), jnp.float32),
    )(dp0, dp1)


def _sc_alpha(ex, rec, dst):
    B = 40
    EPT = E // (NC * NS)
    NB = EPT // B

    @functools.partial(
        pl.kernel,
        out_type=jax.ShapeDtypeStruct((E, HP), jnp.float32),
        mesh=plsc.VectorSubcoreMesh(**_MESH),
        scratch_types=(
            pltpu.VMEM((B,), jnp.int32),
            pltpu.VMEM((B, HP), jnp.float32),
            pltpu.VMEM((B, D), jnp.float32),
            pltpu.VMEM((B, HP), jnp.float32),
        ),
    )
    def k(ex_hbm, rec_hbm, dst_hbm, al_hbm, dstb, exs, grec, alb):
        c = lax.axis_index("c")
        s = lax.axis_index("s")
        base = c * (E // NC) + s * EPT

        def batch(i, carry):
            eb = base + i * B
            pltpu.sync_copy(dst_hbm.at[pl.ds(eb, B)], dstb)
            pltpu.sync_copy(ex_hbm.at[pl.ds(eb, B)], exs)
            pltpu.sync_copy(rec_hbm.at[dstb], grec)
            for r in range(B):
                alb[r, :] = exs[r, :] * grec[r, pl.ds(0, 16)]
            pltpu.sync_copy(alb, al_hbm.at[pl.ds(eb, B)])
            return carry

        lax.fori_loop(0, NB, batch, 0)

    return k(ex, rec, dst)


def _sc_msg0(h0f, ex, src, dst):
    B = 80
    EPT = E // NS
    NB = EPT // B
    HPC = H // NC

    @functools.partial(
        pl.kernel,
        out_type=jax.ShapeDtypeStruct((H * N, D), jnp.float32),
        mesh=plsc.VectorSubcoreMesh(**_MESH),
        scratch_types=(
            pltpu.VMEM((B,), jnp.int32),
            pltpu.VMEM((B,), jnp.int32),
            pltpu.VMEM((B,), jnp.int32),
            pltpu.VMEM((B, HP), jnp.float32),
            pltpu.VMEM((B, D), jnp.float32),
            pltpu.VMEM((CH, D), jnp.float32),
            pltpu.VMEM_SHARED((N, D), jnp.float32),
        ),
    )
    def k(h0_hbm, ex_hbm, src_hbm, dst_hbm, out_hbm,
          srcb, srcb2, dstb, ab, gbuf, zb, acc):
        c = lax.axis_index("c")
        s = lax.axis_index("s")
        zero = jnp.zeros((16,), jnp.float32)

        def zrow(i, carry):
            for q in range(D // 16):
                zb[i, pl.ds(q * 16, 16)] = zero
            return carry

        lax.fori_loop(0, CH, zrow, 0)

        for j in range(HPC):
            h = c * HPC + j
            hN = h * N
            hvec = jnp.full((16,), h, jnp.int32)
            _owned_blocks(
                s, lambda b: pltpu.sync_copy(zb, acc.at[pl.ds(b * CH, CH)]))
            plsc.subcore_barrier()

            def batch(i, carry):
                eb = s * EPT + i * B
                pltpu.sync_copy(src_hbm.at[pl.ds(eb, B)], srcb)
                pltpu.sync_copy(dst_hbm.at[pl.ds(eb, B)], dstb)
                pltpu.sync_copy(ex_hbm.at[pl.ds(eb, B)], ab)
                for q in range(B // 16):
                    srcb2[pl.ds(q * 16, 16)] = srcb[pl.ds(q * 16, 16)] + hN
                pltpu.sync_copy(h0_hbm.at[srcb2], gbuf)

                def row(r, carry2):
                    av = ab[r, :]
                    a16 = av.at[hvec].get(mode="promise_in_bounds")
                    for q in range(D // 16):
                        gbuf[r, pl.ds(q * 16, 16)] = (
                            gbuf[r, pl.ds(q * 16, 16)] * a16)
                    return carry2

                lax.fori_loop(0, B, row, 0)
                pltpu.sync_copy(gbuf, acc.at[dstb], add=True)
                return carry

            lax.fori_loop(0, NB, batch, 0)
            plsc.subcore_barrier()

            def wb(b):
                r0 = b * CH
                pltpu.sync_copy(acc.at[pl.ds(r0, CH)],
                                out_hbm.at[pl.ds(hN + r0, CH)])

            _owned_blocks(s, wb)
            plsc.subcore_barrier()

    return k(h0f, ex, src, dst)


def _sc_msg1(h1, ex1, src, dst):
    B = 40
    EPT = E // (NC * NS)
    NB = EPT // B

    @functools.partial(
        pl.kernel,
        out_type=(
            jax.ShapeDtypeStruct((N, D), jnp.float32),
            jax.ShapeDtypeStruct((N, D), jnp.float32),
        ),
        mesh=plsc.VectorSubcoreMesh(**_MESH),
        scratch_types=(
            pltpu.VMEM((B,), jnp.int32),
            pltpu.VMEM((B,), jnp.int32),
            pltpu.VMEM((B, HP), jnp.float32),
            pltpu.VMEM((B, D), jnp.float32),
            pltpu.VMEM((CH, D), jnp.float32),
            pltpu.VMEM_SHARED((N, D), jnp.float32),
        ),
    )
    def k(h1_hbm, ex_hbm, src_hbm, dst_hbm, p0_hbm, p1_hbm,
          srcb, dstb, ab, gbuf, zb, acc):
        c = lax.axis_index("c")
        s = lax.axis_index("s")
        zero = jnp.zeros((16,), jnp.float32)

        def zrow(i, carry):
            for q in range(D // 16):
                zb[i, pl.ds(q * 16, 16)] = zero
            return carry

        lax.fori_loop(0, CH, zrow, 0)
        _owned_blocks(s, lambda b: pltpu.sync_copy(zb, acc.at[pl.ds(b * CH, CH)]))
        plsc.subcore_barrier()

        base = c * (E // NC) + s * EPT

        def batch(i, carry):
            eb = base + i * B
            pltpu.sync_copy(src_hbm.at[pl.ds(eb, B)], srcb)
            pltpu.sync_copy(dst_hbm.at[pl.ds(eb, B)], dstb)
            pltpu.sync_copy(ex_hbm.at[pl.ds(eb, B)], ab)
            pltpu.sync_copy(h1_hbm.at[srcb], gbuf)

            def row(r, carry2):
                av = ab[r, :]
                a = av[0]
                for q in range(D // 16):
                    gbuf[r, pl.ds(q * 16, 16)] = gbuf[r, pl.ds(q * 16, 16)] * a
                return carry2

            lax.fori_loop(0, B, row, 0)
            pltpu.sync_copy(gbuf, acc.at[dstb], add=True)
            return carry

        lax.fori_loop(0, NB, batch, 0)
        plsc.subcore_barrier()

        def wb(b):
            r0 = b * CH

            @pl.when(c == 0)
            def _():
                pltpu.sync_copy(acc.at[pl.ds(r0, CH)], p0_hbm.at[pl.ds(r0, CH)])

            @pl.when(c == 1)
            def _():
                pltpu.sync_copy(acc.at[pl.ds(r0, CH)], p1_hbm.at[pl.ds(r0, CH)])

        _owned_blocks(s, wb)

    return k(h1, ex1, src, dst)


def _merge_body(p0_ref, p1_ref, dq0_ref, dq1_ref, o_ref):
    den = dq0_ref[:, 0:1] + dq1_ref[:, 0:1] + 1e-16
    o_ref[...] = (p0_ref[...] + p1_ref[...]) / den


def _merge(p0, p1, dq0, dq1):
    return pl.pallas_call(
        _merge_body,
        grid=(N // RB,),
        in_specs=[
            pl.BlockSpec((RB, D), lambda i: (i, 0)),
            pl.BlockSpec((RB, D), lambda i: (i, 0)),
            pl.BlockSpec((RB, D), lambda i: (i, 0)),
            pl.BlockSpec((RB, D), lambda i: (i, 0)),
        ],
        out_specs=pl.BlockSpec((RB, D), lambda i: (i, 0)),
        out_shape=jax.ShapeDtypeStruct((N, D), jnp.float32),
    )(p0, p1, dq0, dq1)


_BE = 8000


def _amean_body(a_ref, o_ref):
    a = a_ref[...]
    o_ref[...] = jnp.sum(a[:, :H], axis=1, keepdims=True) * (1.0 / H)


def _amean(alpha):
    return pl.pallas_call(
        _amean_body,
        grid=(E // _BE,),
        in_specs=[pl.BlockSpec((_BE, HP), lambda i: (i, 0))],
        out_specs=pl.BlockSpec((_BE, 1), lambda i: (i, 0)),
        out_shape=jax.ShapeDtypeStruct((E, 1), jnp.float32),
    )(alpha)


def kernel(reg_info, inputs, edge_index, W0, a_src0, a_dst0, W1, a_src1, a_dst1):
    x = inputs[0]
    src = edge_index[0]
    dst = edge_index[1]

    onehot = jnp.eye(HP, dtype=jnp.float32)[:H]
    Ap_src = (a_src0[:, :, None] * onehot[:, None, :]).reshape(H * D, HP)
    Ap_dst = (a_dst0[:, :, None] * onehot[:, None, :]).reshape(H * D, HP)
    Apad = jnp.concatenate(
        [Ap_src, Ap_dst, jnp.zeros((H * D, D - 2 * HP), jnp.float32)], axis=1)
    A1pad = jnp.concatenate(
        [jnp.broadcast_to(a_src1.reshape(D, 1), (D, HP)),
         jnp.broadcast_to(a_dst1.reshape(D, 1), (D, HP)),
         jnp.zeros((D, D - 2 * HP), jnp.float32)], axis=1)

    h0, asad0 = _dense0(x, W0, Apad)
    h0f = h0.reshape(H * N, D)

    ex0, dp0, dp1 = _sc_logits(asad0, src, dst)
    out0f = _sc_msg0(h0f, ex0, src, dst)
    out0 = out0f.reshape(H, N, D)

    rec0 = _recip(dp0, dp1)
    alpha0 = _sc_alpha(ex0, rec0, dst)

    h1, asad1 = _dense1(out0, dp0, dp1, W1, A1pad)
    ex1, dq0, dq1 = _sc_logits(asad1, src, dst)
    p0, p1 = _sc_msg1(h1, ex1, src, dst)

    out = _merge(p0, p1, dq0, dq1)
    anorm = _amean(alpha0)
    return out.reshape(1, N, D), anorm.reshape(E)


# speedup vs baseline: 13.6995x; 1.9707x over previous
import functools

import jax
import jax.numpy as jnp
from jax import lax
from jax.experimental import pallas as pl
from jax.experimental.pallas import tpu as pltpu
from jax.experimental.pallas import tpu_sc as plsc

N = 10000
E = 160000
D = 128
H = 8
HP = 16
NC = 2
NS = 16
CH = 200
NBLK = N // CH
NBI = (NBLK + NS - 1) // NS
ZB = 40
RB = 1000

_MESH = dict(core_axis_name="c", subcore_axis_name="s", num_cores=NC,
             num_subcores=NS)


def _owned_blocks(s, body):
    def it(i, carry):
        b = i * NS + s

        @pl.when(b < NBLK)
        def _():
            body(b)

        return carry

    lax.fori_loop(0, NBI, it, 0)


def _zero_blocks(s, zb, acc):
    def z(b):
        for k in range(CH // ZB):
            pltpu.sync_copy(zb, acc.at[pl.ds(b * CH + k * ZB, ZB)])

    _owned_blocks(s, z)


def _dense0_body(x_ref, w_ref, apad_ref, h0_ref, asad_ref):
    xb = x_ref[...]
    hs = []
    for h in range(H):
        ph = jnp.dot(xb, w_ref[:, h * D:(h + 1) * D],
                     preferred_element_type=jnp.float32)
        h0_ref[h] = ph
        hs.append(ph)
    hb = jnp.concatenate(hs, axis=1)
    asad_ref[...] = jnp.dot(hb, apad_ref[...], preferred_element_type=jnp.float32)


def _dense0(x, W0, Apad):
    return pl.pallas_call(
        _dense0_body,
        grid=(N // RB,),
        in_specs=[
            pl.BlockSpec((RB, D), lambda i: (i, 0)),
            pl.BlockSpec((D, H * D), lambda i: (0, 0)),
            pl.BlockSpec((H * D, D), lambda i: (0, 0)),
        ],
        out_specs=[
            pl.BlockSpec((H, RB, D), lambda i: (0, i, 0)),
            pl.BlockSpec((RB, D), lambda i: (i, 0)),
        ],
        out_shape=[
            jax.ShapeDtypeStruct((H, N, D), jnp.float32),
            jax.ShapeDtypeStruct((N, D), jnp.float32),
        ],
    )(x, W0, Apad)


def _dense1_body(o_ref, dp0_ref, dp1_ref, w1_ref, a1pad_ref,
                 h1_ref, asad_ref):
    acc = jnp.zeros((RB, D), jnp.float32)
    for h in range(H):
        den = dp0_ref[:, h:h + 1] + dp1_ref[:, h:h + 1] + 1e-16
        xh = o_ref[h] / den
        xh = jnp.where(xh > 0, xh, jnp.exp(xh) - 1.0)
        acc = acc + jnp.dot(xh, w1_ref[h * D:(h + 1) * D, :],
                            preferred_element_type=jnp.float32)
    h1_ref[...] = acc
    asad_ref[...] = jnp.dot(acc, a1pad_ref[...], preferred_element_type=jnp.float32)


def _dense1(out0, dp0, dp1, W1, A1pad):
    return pl.pallas_call(
        _dense1_body,
        grid=(N // RB,),
        in_specs=[
            pl.BlockSpec((H, RB, D), lambda i: (0, i, 0)),
            pl.BlockSpec((RB, D), lambda i: (i, 0)),
            pl.BlockSpec((RB, D), lambda i: (i, 0)),
            pl.BlockSpec((H * D, D), lambda i: (0, 0)),
            pl.BlockSpec((D, D), lambda i: (0, 0)),
        ],
        out_specs=[
            pl.BlockSpec((RB, D), lambda i: (i, 0)),
            pl.BlockSpec((RB, D), lambda i: (i, 0)),
        ],
        out_shape=[
            jax.ShapeDtypeStruct((N, D), jnp.float32),
            jax.ShapeDtypeStruct((N, D), jnp.float32),
        ],
    )(out0, dp0, dp1, W1, A1pad)


def _sc_logits(asad, src, dst):
    B = 40
    EPT = E // (NC * NS)
    NB = EPT // B
    assert NB % 2 == 1

    @functools.partial(
        pl.kernel,
        out_type=(
            jax.ShapeDtypeStruct((E, HP), jnp.float32),
            jax.ShapeDtypeStruct((N, D), jnp.float32),
            jax.ShapeDtypeStruct((N, D), jnp.float32),
        ),
        mesh=plsc.VectorSubcoreMesh(**_MESH),
        scratch_types=(
            (pltpu.VMEM((B,), jnp.int32),) * 2,
            (pltpu.VMEM((B,), jnp.int32),) * 2,
            (pltpu.VMEM((B, D), jnp.float32),) * 2,
            (pltpu.VMEM((B, D), jnp.float32),) * 2,
            pltpu.VMEM((B, D), jnp.float32),
            pltpu.VMEM((B, HP), jnp.float32),
            pltpu.VMEM((ZB, D), jnp.float32),
            pltpu.VMEM_SHARED((N, D), jnp.float32),
            pltpu.SemaphoreType.DMA,
            (pltpu.SemaphoreType.DMA,) * 2,
        ),
    )
    def k(asad_hbm, src_hbm, dst_hbm, ex_hbm, dp0_hbm, dp1_hbm,
          srcb, dstb, gA, gB, exb, exs, zb, acc, semM, semG):
        c = lax.axis_index("c")
        s = lax.axis_index("s")
        zero = jnp.zeros((16,), jnp.float32)

        def zrow(i, carry):
            for q in range(D // 16):
                zb[i, pl.ds(q * 16, 16)] = zero
            return carry

        lax.fori_loop(0, ZB, zrow, 0)
        _zero_blocks(s, zb, acc)

        def zrow2(i, carry):
            for q in range(D // 16):
                exb[i, pl.ds(q * 16, 16)] = zero
            return carry

        lax.fori_loop(0, B, zrow2, 0)
        plsc.subcore_barrier()

        base = c * (E // NC) + s * EPT

        def fire(b, u):
            eb = base + b * B
            c1 = pltpu.async_copy(src_hbm.at[pl.ds(eb, B)], srcb[u], semM)
            c2 = pltpu.async_copy(dst_hbm.at[pl.ds(eb, B)], dstb[u], semM)
            c1.wait()
            c2.wait()
            pltpu.async_copy(asad_hbm.at[srcb[u]], gA[u], semG[u])
            pltpu.async_copy(asad_hbm.at[dstb[u]], gB[u], semG[u])

        def process(b, u):
            eb = base + b * B
            pltpu.make_async_copy(asad_hbm.at[srcb[u]], gA[u], semG[u]).wait()
            pltpu.make_async_copy(asad_hbm.at[dstb[u]], gB[u], semG[u]).wait()
            for r in range(B):
                e = gA[u][r, pl.ds(0, 16)] + gB[u][r, pl.ds(16, 16)]
                e = jnp.where(e > 0.0, e, 0.2 * e)
                ex = jnp.exp(e)
                exb[r, pl.ds(0, 16)] = ex
                exs[r, :] = ex
            pltpu.sync_copy(exs, ex_hbm.at[pl.ds(eb, B)])
            pltpu.sync_copy(exb, acc.at[dstb[u]], add=True)

        fire(0, 0)

        def pair(i, carry):
            b0 = 2 * i
            fire(b0 + 1, 1)
            process(b0, 0)
            fire(b0 + 2, 0)
            process(b0 + 1, 1)
            return carry

        lax.fori_loop(0, NB // 2, pair, 0)
        process(NB - 1, 0)
        plsc.subcore_barrier()

        def wb(b):
            r0 = b * CH

            @pl.when(c == 0)
            def _():
                pltpu.sync_copy(acc.at[pl.ds(r0, CH)], dp0_hbm.at[pl.ds(r0, CH)])

            @pl.when(c == 1)
            def _():
                pltpu.sync_copy(acc.at[pl.ds(r0, CH)], dp1_hbm.at[pl.ds(r0, CH)])

        _owned_blocks(s, wb)

    return k(asad, src, dst)


def _recip_body(dp0_ref, dp1_ref, o_ref):
    o_ref[...] = 1.0 / (dp0_ref[...] + dp1_ref[...] + 1e-16)


def _recip(dp0, dp1):
    return pl.pallas_call(
        _recip_body,
        grid=(N // RB,),
        in_specs=[
            pl.BlockSpec((RB, D), lambda i: (i, 0)),
            pl.BlockSpec((RB, D), lambda i: (i, 0)),
        ],
        out_specs=pl.BlockSpec((RB, D), lambda i: (i, 0)),
        out_shape=jax.ShapeDtypeStruct((N, D), jnp.float32),
    )(dp0, dp1)


def _sc_alpha(ex, rec, dst):
    B = 40
    EPT = E // (NC * NS)
    NB = EPT // B

    assert NB % 2 == 1

    @functools.partial(
        pl.kernel,
        out_type=jax.ShapeDtypeStruct((E, HP), jnp.float32),
        mesh=plsc.VectorSubcoreMesh(**_MESH),
        scratch_types=(
            (pltpu.VMEM((B,), jnp.int32),) * 2,
            (pltpu.VMEM((B, HP), jnp.float32),) * 2,
            (pltpu.VMEM((B, D), jnp.float32),) * 2,
            pltpu.VMEM((B, HP), jnp.float32),
            pltpu.SemaphoreType.DMA,
            (pltpu.SemaphoreType.DMA,) * 2,
        ),
    )
    def k(ex_hbm, rec_hbm, dst_hbm, al_hbm, dstb, exs, grec, alb, semM, semG):
        c = lax.axis_index("c")
        s = lax.axis_index("s")
        base = c * (E // NC) + s * EPT

        def fire(b, u):
            eb = base + b * B
            c1 = pltpu.async_copy(dst_hbm.at[pl.ds(eb, B)], dstb[u], semM)
            c2 = pltpu.async_copy(ex_hbm.at[pl.ds(eb, B)], exs[u], semM)
            c1.wait()
            c2.wait()
            pltpu.async_copy(rec_hbm.at[dstb[u]], grec[u], semG[u])

        def process(b, u):
            eb = base + b * B
            pltpu.make_async_copy(rec_hbm.at[dstb[u]], grec[u], semG[u]).wait()
            for r in range(B):
                alb[r, :] = exs[u][r, :] * grec[u][r, pl.ds(0, 16)]
            pltpu.sync_copy(alb, al_hbm.at[pl.ds(eb, B)])

        fire(0, 0)

        def pair(i, carry):
            b0 = 2 * i
            fire(b0 + 1, 1)
            process(b0, 0)
            fire(b0 + 2, 0)
            process(b0 + 1, 1)
            return carry

        lax.fori_loop(0, NB // 2, pair, 0)
        process(NB - 1, 0)

    return k(ex, rec, dst)


def _sc_msg0(h0f, ex, src, dst):
    B = 80
    EPT = E // NS
    NB = EPT // B
    HPC = H // NC

    assert NB % 2 == 1

    @functools.partial(
        pl.kernel,
        out_type=jax.ShapeDtypeStruct((H * N, D), jnp.float32),
        mesh=plsc.VectorSubcoreMesh(**_MESH),
        scratch_types=(
            (pltpu.VMEM((B,), jnp.int32),) * 2,
            (pltpu.VMEM((B,), jnp.int32),) * 2,
            (pltpu.VMEM((B,), jnp.int32),) * 2,
            (pltpu.VMEM((B, HP), jnp.float32),) * 2,
            (pltpu.VMEM((B, D), jnp.float32),) * 2,
            pltpu.VMEM((ZB, D), jnp.float32),
            pltpu.VMEM_SHARED((N, D), jnp.float32),
            pltpu.SemaphoreType.DMA,
            (pltpu.SemaphoreType.DMA,) * 2,
        ),
    )
    def k(h0_hbm, ex_hbm, src_hbm, dst_hbm, out_hbm,
          srcb, srcb2, dstb, ab, gbuf, zb, acc, semM, semG):
        c = lax.axis_index("c")
        s = lax.axis_index("s")
        zero = jnp.zeros((16,), jnp.float32)

        def zrow(i, carry):
            for q in range(D // 16):
                zb[i, pl.ds(q * 16, 16)] = zero
            return carry

        lax.fori_loop(0, ZB, zrow, 0)

        for j in range(HPC):
            h = c * HPC + j
            hN = h * N
            hvec = jnp.full((16,), h, jnp.int32)
            _zero_blocks(s, zb, acc)
            plsc.subcore_barrier()

            def fire(b, u):
                eb = s * EPT + b * B
                c1 = pltpu.async_copy(src_hbm.at[pl.ds(eb, B)], srcb[u], semM)
                c2 = pltpu.async_copy(dst_hbm.at[pl.ds(eb, B)], dstb[u], semM)
                c3 = pltpu.async_copy(ex_hbm.at[pl.ds(eb, B)], ab[u], semM)
                c1.wait()
                c2.wait()
                c3.wait()
                for q in range(B // 16):
                    srcb2[u][pl.ds(q * 16, 16)] = srcb[u][pl.ds(q * 16, 16)] + hN
                pltpu.async_copy(h0_hbm.at[srcb2[u]], gbuf[u], semG[u])

            def process(u):
                pltpu.make_async_copy(h0_hbm.at[srcb2[u]], gbuf[u],
                                      semG[u]).wait()

                def row(r, carry2):
                    av = ab[u][r, :]
                    a16 = av.at[hvec].get(mode="promise_in_bounds")
                    for q in range(D // 16):
                        gbuf[u][r, pl.ds(q * 16, 16)] = (
                            gbuf[u][r, pl.ds(q * 16, 16)] * a16)
                    return carry2

                lax.fori_loop(0, B, row, 0)
                pltpu.sync_copy(gbuf[u], acc.at[dstb[u]], add=True)

            fire(0, 0)

            def pair(i, carry):
                b0 = 2 * i
                fire(b0 + 1, 1)
                process(0)
                fire(b0 + 2, 0)
                process(1)
                return carry

            lax.fori_loop(0, NB // 2, pair, 0)
            process(0)
            plsc.subcore_barrier()

            def wb(b):
                r0 = b * CH
                pltpu.sync_copy(acc.at[pl.ds(r0, CH)],
                                out_hbm.at[pl.ds(hN + r0, CH)])

            _owned_blocks(s, wb)
            plsc.subcore_barrier()

    return k(h0f, ex, src, dst)


def _sc_msg1(h1, ex1, src, dst):
    B = 40
    EPT = E // (NC * NS)
    NB = EPT // B

    @functools.partial(
        pl.kernel,
        out_type=(
            jax.ShapeDtypeStruct((N, D), jnp.float32),
            jax.ShapeDtypeStruct((N, D), jnp.float32),
        ),
        mesh=plsc.VectorSubcoreMesh(**_MESH),
        scratch_types=(
            (pltpu.VMEM((B,), jnp.int32),) * 2,
            (pltpu.VMEM((B,), jnp.int32),) * 2,
            (pltpu.VMEM((B, HP), jnp.float32),) * 2,
            (pltpu.VMEM((B, D), jnp.float32),) * 2,
            pltpu.VMEM((ZB, D), jnp.float32),
            pltpu.VMEM_SHARED((N, D), jnp.float32),
            pltpu.SemaphoreType.DMA,
            (pltpu.SemaphoreType.DMA,) * 2,
        ),
    )
    def k(h1_hbm, ex_hbm, src_hbm, dst_hbm, p0_hbm, p1_hbm,
          srcb, dstb, ab, gbuf, zb, acc, semM, semG):
        c = lax.axis_index("c")
        s = lax.axis_index("s")
        zero = jnp.zeros((16,), jnp.float32)

        def zrow(i, carry):
            for q in range(D // 16):
                zb[i, pl.ds(q * 16, 16)] = zero
            return carry

        lax.fori_loop(0, ZB, zrow, 0)
        _zero_blocks(s, zb, acc)
        plsc.subcore_barrier()

        base = c * (E // NC) + s * EPT

        def fire(b, u):
            eb = base + b * B
            c1 = pltpu.async_copy(src_hbm.at[pl.ds(eb, B)], srcb[u], semM)
            c2 = pltpu.async_copy(dst_hbm.at[pl.ds(eb, B)], dstb[u], semM)
            c3 = pltpu.async_copy(ex_hbm.at[pl.ds(eb, B)], ab[u], semM)
            c1.wait()
            c2.wait()
            c3.wait()
            pltpu.async_copy(h1_hbm.at[srcb[u]], gbuf[u], semG[u])

        def process(u):
            pltpu.make_async_copy(h1_hbm.at[srcb[u]], gbuf[u], semG[u]).wait()

            def row(r, carry2):
                av = ab[u][r, :]
                a = av[0]
                for q in range(D // 16):
                    gbuf[u][r, pl.ds(q * 16, 16)] = (
                        gbuf[u][r, pl.ds(q * 16, 16)] * a)
                return carry2

            lax.fori_loop(0, B, row, 0)
            pltpu.sync_copy(gbuf[u], acc.at[dstb[u]], add=True)

        fire(0, 0)

        def pair(i, carry):
            b0 = 2 * i
            fire(b0 + 1, 1)
            process(0)
            fire(b0 + 2, 0)
            process(1)
            return carry

        lax.fori_loop(0, NB // 2, pair, 0)
        process(0)
        plsc.subcore_barrier()

        def wb(b):
            r0 = b * CH

            @pl.when(c == 0)
            def _():
                pltpu.sync_copy(acc.at[pl.ds(r0, CH)], p0_hbm.at[pl.ds(r0, CH)])

            @pl.when(c == 1)
            def _():
                pltpu.sync_copy(acc.at[pl.ds(r0, CH)], p1_hbm.at[pl.ds(r0, CH)])

        _owned_blocks(s, wb)

    return k(h1, ex1, src, dst)


def _merge_body(p0_ref, p1_ref, dq0_ref, dq1_ref, o_ref):
    den = dq0_ref[:, 0:1] + dq1_ref[:, 0:1] + 1e-16
    o_ref[...] = (p0_ref[...] + p1_ref[...]) / den


def _merge(p0, p1, dq0, dq1):
    return pl.pallas_call(
        _merge_body,
        grid=(N // RB,),
        in_specs=[
            pl.BlockSpec((RB, D), lambda i: (i, 0)),
            pl.BlockSpec((RB, D), lambda i: (i, 0)),
            pl.BlockSpec((RB, D), lambda i: (i, 0)),
            pl.BlockSpec((RB, D), lambda i: (i, 0)),
        ],
        out_specs=pl.BlockSpec((RB, D), lambda i: (i, 0)),
        out_shape=jax.ShapeDtypeStruct((N, D), jnp.float32),
    )(p0, p1, dq0, dq1)


_BE = 8000


def _amean_body(a_ref, o_ref):
    a = a_ref[...]
    o_ref[...] = jnp.sum(a[:, :H], axis=1, keepdims=True) * (1.0 / H)


def _amean(alpha):
    return pl.pallas_call(
        _amean_body,
        grid=(E // _BE,),
        in_specs=[pl.BlockSpec((_BE, HP), lambda i: (i, 0))],
        out_specs=pl.BlockSpec((_BE, 1), lambda i: (i, 0)),
        out_shape=jax.ShapeDtypeStruct((E, 1), jnp.float32),
    )(alpha)


def kernel(reg_info, inputs, edge_index, W0, a_src0, a_dst0, W1, a_src1, a_dst1):
    x = inputs[0]
    src = edge_index[0]
    dst = edge_index[1]

    onehot = jnp.eye(HP, dtype=jnp.float32)[:H]
    Ap_src = (a_src0[:, :, None] * onehot[:, None, :]).reshape(H * D, HP)
    Ap_dst = (a_dst0[:, :, None] * onehot[:, None, :]).reshape(H * D, HP)
    Apad = jnp.concatenate(
        [Ap_src, Ap_dst, jnp.zeros((H * D, D - 2 * HP), jnp.float32)], axis=1)
    A1pad = jnp.concatenate(
        [jnp.broadcast_to(a_src1.reshape(D, 1), (D, HP)),
         jnp.broadcast_to(a_dst1.reshape(D, 1), (D, HP)),
         jnp.zeros((D, D - 2 * HP), jnp.float32)], axis=1)

    h0, asad0 = _dense0(x, W0, Apad)
    h0f = h0.reshape(H * N, D)

    ex0, dp0, dp1 = _sc_logits(asad0, src, dst)
    out0f = _sc_msg0(h0f, ex0, src, dst)
    out0 = out0f.reshape(H, N, D)

    rec0 = _recip(dp0, dp1)
    alpha0 = _sc_alpha(ex0, rec0, dst)

    h1, asad1 = _dense1(out0, dp0, dp1, W1, A1pad)
    ex1, dq0, dq1 = _sc_logits(asad1, src, dst)
    p0, p1 = _sc_msg1(h1, ex1, src, dst)

    out = _merge(p0, p1, dq0, dq1)
    anorm = _amean(alpha0)
    return out.reshape(1, N, D), anorm.reshape(E)


# speedup vs baseline: 17.4497x; 1.2737x over previous
import functools

import jax
import jax.numpy as jnp
from jax import lax
from jax.experimental import pallas as pl
from jax.experimental.pallas import tpu as pltpu
from jax.experimental.pallas import tpu_sc as plsc

N = 10000
E = 160000
D = 128
H = 8
HP = 16
NC = 2
NS = 16
CH = 200
NBLK = N // CH
NBI = (NBLK + NS - 1) // NS
ZB = 40
RB = 1000

_MESH = dict(core_axis_name="c", subcore_axis_name="s", num_cores=NC,
             num_subcores=NS)


def _owned_blocks(s, body):
    def it(i, carry):
        b = i * NS + s

        @pl.when(b < NBLK)
        def _():
            body(b)

        return carry

    lax.fori_loop(0, NBI, it, 0)


def _zero_blocks(s, zb, acc):
    def z(b):
        for k in range(CH // ZB):
            pltpu.sync_copy(zb, acc.at[pl.ds(b * CH + k * ZB, ZB)])

    _owned_blocks(s, z)


def _dense0_body(x_ref, w_ref, apad_ref, h0_ref, asad_ref):
    xb = x_ref[...]
    hs = []
    for h in range(H):
        ph = jnp.dot(xb, w_ref[:, h * D:(h + 1) * D],
                     preferred_element_type=jnp.float32)
        h0_ref[h] = ph
        hs.append(ph)
    hb = jnp.concatenate(hs, axis=1)
    asad_ref[...] = jnp.dot(hb, apad_ref[...], preferred_element_type=jnp.float32)


def _dense0(x, W0, Apad):
    return pl.pallas_call(
        _dense0_body,
        grid=(N // RB,),
        in_specs=[
            pl.BlockSpec((RB, D), lambda i: (i, 0)),
            pl.BlockSpec((D, H * D), lambda i: (0, 0)),
            pl.BlockSpec((H * D, D), lambda i: (0, 0)),
        ],
        out_specs=[
            pl.BlockSpec((H, RB, D), lambda i: (0, i, 0)),
            pl.BlockSpec((RB, D), lambda i: (i, 0)),
        ],
        out_shape=[
            jax.ShapeDtypeStruct((H, N, D), jnp.float32),
            jax.ShapeDtypeStruct((N, D), jnp.float32),
        ],
    )(x, W0, Apad)


def _dense1_body(o_ref, dp0_ref, dp1_ref, w1_ref, a1pad_ref,
                 h1_ref, asad_ref):
    acc = jnp.zeros((RB, D), jnp.float32)
    for h in range(H):
        den = dp0_ref[:, h:h + 1] + dp1_ref[:, h:h + 1] + 1e-16
        xh = o_ref[h] / den
        xh = jnp.where(xh > 0, xh, jnp.exp(xh) - 1.0)
        acc = acc + jnp.dot(xh, w1_ref[h * D:(h + 1) * D, :],
                            preferred_element_type=jnp.float32)
    h1_ref[...] = acc
    asad_ref[...] = jnp.dot(acc, a1pad_ref[...], preferred_element_type=jnp.float32)


def _dense1(out0, dp0, dp1, W1, A1pad):
    return pl.pallas_call(
        _dense1_body,
        grid=(N // RB,),
        in_specs=[
            pl.BlockSpec((H, RB, D), lambda i: (0, i, 0)),
            pl.BlockSpec((RB, D), lambda i: (i, 0)),
            pl.BlockSpec((RB, D), lambda i: (i, 0)),
            pl.BlockSpec((H * D, D), lambda i: (0, 0)),
            pl.BlockSpec((D, D), lambda i: (0, 0)),
        ],
        out_specs=[
            pl.BlockSpec((RB, D), lambda i: (i, 0)),
            pl.BlockSpec((RB, D), lambda i: (i, 0)),
        ],
        out_shape=[
            jax.ShapeDtypeStruct((N, D), jnp.float32),
            jax.ShapeDtypeStruct((N, D), jnp.float32),
        ],
    )(out0, dp0, dp1, W1, A1pad)


def _sc_logits(asad, src, dst):
    B = 40
    EPT = E // (NC * NS)
    NB = EPT // B
    assert NB % 2 == 1

    @functools.partial(
        pl.kernel,
        out_type=(
            jax.ShapeDtypeStruct((E * HP,), jnp.float32),
            jax.ShapeDtypeStruct((N, D), jnp.float32),
            jax.ShapeDtypeStruct((N, D), jnp.float32),
        ),
        mesh=plsc.VectorSubcoreMesh(**_MESH),
        scratch_types=(
            (pltpu.VMEM((B,), jnp.int32),) * 2,
            (pltpu.VMEM((B,), jnp.int32),) * 2,
            (pltpu.VMEM((B, D), jnp.float32),) * 2,
            (pltpu.VMEM((B, D), jnp.float32),) * 2,
            pltpu.VMEM((B, D), jnp.float32),
            pltpu.VMEM((B * HP,), jnp.float32),
            pltpu.VMEM((ZB, D), jnp.float32),
            pltpu.VMEM_SHARED((N, D), jnp.float32),
            pltpu.SemaphoreType.DMA,
            (pltpu.SemaphoreType.DMA,) * 2,
        ),
    )
    def k(asad_hbm, src_hbm, dst_hbm, ex_hbm, dp0_hbm, dp1_hbm,
          srcb, dstb, gA, gB, exb, exs, zb, acc, semM, semG):
        c = lax.axis_index("c")
        s = lax.axis_index("s")
        zero = jnp.zeros((16,), jnp.float32)

        def zrow(i, carry):
            for q in range(D // 16):
                zb[i, pl.ds(q * 16, 16)] = zero
            return carry

        lax.fori_loop(0, ZB, zrow, 0)
        _zero_blocks(s, zb, acc)

        def zrow2(i, carry):
            for q in range(D // 16):
                exb[i, pl.ds(q * 16, 16)] = zero
            return carry

        lax.fori_loop(0, B, zrow2, 0)
        plsc.subcore_barrier()

        base = c * (E // NC) + s * EPT

        def fire(b, u):
            eb = base + b * B
            c1 = pltpu.async_copy(src_hbm.at[pl.ds(eb, B)], srcb[u], semM)
            c2 = pltpu.async_copy(dst_hbm.at[pl.ds(eb, B)], dstb[u], semM)
            c1.wait()
            c2.wait()
            pltpu.async_copy(asad_hbm.at[srcb[u]], gA[u], semG[u])
            pltpu.async_copy(asad_hbm.at[dstb[u]], gB[u], semG[u])

        def process(b, u):
            eb = base + b * B
            pltpu.make_async_copy(asad_hbm.at[srcb[u]], gA[u], semG[u]).wait()
            pltpu.make_async_copy(asad_hbm.at[dstb[u]], gB[u], semG[u]).wait()
            for r in range(B):
                e = gA[u][r, pl.ds(0, 16)] + gB[u][r, pl.ds(16, 16)]
                e = jnp.where(e > 0.0, e, 0.2 * e)
                ex = jnp.exp(e)
                exb[r, pl.ds(0, 16)] = ex
                exs[pl.ds(r * HP, 16)] = ex
            pltpu.sync_copy(exs, ex_hbm.at[pl.ds(eb * HP, B * HP)])
            pltpu.sync_copy(exb, acc.at[dstb[u]], add=True)

        fire(0, 0)

        def pair(i, carry):
            b0 = 2 * i
            fire(b0 + 1, 1)
            process(b0, 0)
            fire(b0 + 2, 0)
            process(b0 + 1, 1)
            return carry

        lax.fori_loop(0, NB // 2, pair, 0)
        process(NB - 1, 0)
        plsc.subcore_barrier()

        def wb(b):
            r0 = b * CH

            @pl.when(c == 0)
            def _():
                pltpu.sync_copy(acc.at[pl.ds(r0, CH)], dp0_hbm.at[pl.ds(r0, CH)])

            @pl.when(c == 1)
            def _():
                pltpu.sync_copy(acc.at[pl.ds(r0, CH)], dp1_hbm.at[pl.ds(r0, CH)])

        _owned_blocks(s, wb)

    return k(asad, src, dst)


def _recip_body(dp0_ref, dp1_ref, o_ref):
    o_ref[...] = 1.0 / (dp0_ref[...] + dp1_ref[...] + 1e-16)


def _recip(dp0, dp1):
    return pl.pallas_call(
        _recip_body,
        grid=(N // RB,),
        in_specs=[
            pl.BlockSpec((RB, D), lambda i: (i, 0)),
            pl.BlockSpec((RB, D), lambda i: (i, 0)),
        ],
        out_specs=pl.BlockSpec((RB, D), lambda i: (i, 0)),
        out_shape=jax.ShapeDtypeStruct((N, D), jnp.float32),
    )(dp0, dp1)


def _sc_alpha(ex, rec, dst):
    B = 40
    EPT = E // (NC * NS)
    NB = EPT // B

    assert NB % 2 == 1

    @functools.partial(
        pl.kernel,
        out_type=jax.ShapeDtypeStruct((E * HP,), jnp.float32),
        mesh=plsc.VectorSubcoreMesh(**_MESH),
        scratch_types=(
            (pltpu.VMEM((B,), jnp.int32),) * 2,
            (pltpu.VMEM((B * HP,), jnp.float32),) * 2,
            (pltpu.VMEM((B, D), jnp.float32),) * 2,
            pltpu.VMEM((B * HP,), jnp.float32),
            pltpu.SemaphoreType.DMA,
            (pltpu.SemaphoreType.DMA,) * 2,
        ),
    )
    def k(ex_hbm, rec_hbm, dst_hbm, al_hbm, dstb, exs, grec, alb, semM, semG):
        c = lax.axis_index("c")
        s = lax.axis_index("s")
        base = c * (E // NC) + s * EPT

        def fire(b, u):
            eb = base + b * B
            c1 = pltpu.async_copy(dst_hbm.at[pl.ds(eb, B)], dstb[u], semM)
            c2 = pltpu.async_copy(ex_hbm.at[pl.ds(eb * HP, B * HP)], exs[u], semM)
            c1.wait()
            c2.wait()
            pltpu.async_copy(rec_hbm.at[dstb[u]], grec[u], semG[u])

        def process(b, u):
            eb = base + b * B
            pltpu.make_async_copy(rec_hbm.at[dstb[u]], grec[u], semG[u]).wait()
            for r in range(B):
                alb[pl.ds(r * HP, 16)] = (exs[u][pl.ds(r * HP, 16)]
                                          * grec[u][r, pl.ds(0, 16)])
            pltpu.sync_copy(alb, al_hbm.at[pl.ds(eb * HP, B * HP)])

        fire(0, 0)

        def pair(i, carry):
            b0 = 2 * i
            fire(b0 + 1, 1)
            process(b0, 0)
            fire(b0 + 2, 0)
            process(b0 + 1, 1)
            return carry

        lax.fori_loop(0, NB // 2, pair, 0)
        process(NB - 1, 0)

    return k(ex, rec, dst)


def _sc_msg0(h0f, ex, src, dst):
    B = 80
    G = 5
    MB = G * B
    EPT = E // NS
    NB = EPT // B
    HPC = H // NC

    assert NB % 2 == 1 and B % 16 == 0 and NB % G == 0

    @functools.partial(
        pl.kernel,
        out_type=jax.ShapeDtypeStruct((H * N, D), jnp.float32),
        mesh=plsc.VectorSubcoreMesh(**_MESH),
        scratch_types=(
            pltpu.VMEM((MB,), jnp.int32),
            pltpu.VMEM((MB,), jnp.int32),
            pltpu.VMEM((MB * HP,), jnp.float32),
            (pltpu.VMEM((B,), jnp.int32),) * 2,
            (pltpu.VMEM((B,), jnp.int32),) * 2,
            (pltpu.VMEM((B, D), jnp.float32),) * 2,
            pltpu.VMEM((ZB, D), jnp.float32),
            pltpu.VMEM_SHARED((N, D), jnp.float32),
            pltpu.SemaphoreType.DMA,
            (pltpu.SemaphoreType.DMA,) * 2,
            (pltpu.SemaphoreType.DMA,) * 2,
        ),
    )
    def k(h0_hbm, ex_hbm, src_hbm, dst_hbm, out_hbm,
          srcm, dstm, exm, srcb2, dstb, gbuf, zb, acc, semM, semG, semS):
        c = lax.axis_index("c")
        s = lax.axis_index("s")
        zero = jnp.zeros((16,), jnp.float32)
        iota = lax.iota(jnp.int32, 16)
        zi = iota * 0

        def zrow(i, carry):
            for q in range(D // 16):
                zb[i, pl.ds(q * 16, 16)] = zero
            return carry

        lax.fori_loop(0, ZB, zrow, 0)

        for j in range(HPC):
            h = c * HPC + j
            hN = h * N
            hvec = jnp.full((16,), h, jnp.int32)
            _zero_blocks(s, zb, acc)
            plsc.subcore_barrier()

            def fire(b, u):
                kk = lax.rem(b, G)

                @pl.when(kk == 0)
                def _():
                    eb = s * EPT + b * B
                    c1 = pltpu.async_copy(src_hbm.at[pl.ds(eb, MB)], srcm, semM)
                    c2 = pltpu.async_copy(dst_hbm.at[pl.ds(eb, MB)], dstm, semM)
                    c3 = pltpu.async_copy(
                        ex_hbm.at[pl.ds(eb * HP, MB * HP)], exm, semM)
                    c1.wait()
                    c2.wait()
                    c3.wait()

                @pl.when(b >= 2)
                def _():
                    pltpu.make_async_copy(gbuf[u], acc.at[dstb[u]],
                                          semS[u]).wait()

                koff = kk * B
                for q in range(B // 16):
                    srcb2[u][pl.ds(q * 16, 16)] = (
                        srcm[pl.ds(koff + q * 16, 16)] + hN)
                    dstb[u][pl.ds(q * 16, 16)] = dstm[pl.ds(koff + q * 16, 16)]
                pltpu.async_copy(h0_hbm.at[srcb2[u]], gbuf[u], semG[u])

            def process(b, u):
                pltpu.make_async_copy(h0_hbm.at[srcb2[u]], gbuf[u],
                                      semG[u]).wait()
                koff = lax.rem(b, G) * B

                def grp(rg, carry2):
                    rbase = rg * 16
                    for r16 in range(16):
                        av = exm[pl.ds((koff + rbase + r16) * HP, 16)]
                        a = av.at[hvec].get(mode="promise_in_bounds")
                        for q in range(D // 16):
                            gbuf[u][rbase + r16, pl.ds(q * 16, 16)] = (
                                gbuf[u][rbase + r16, pl.ds(q * 16, 16)] * a)
                    return carry2

                lax.fori_loop(0, B // 16, grp, 0)
                pltpu.async_copy(gbuf[u], acc.at[dstb[u]], semS[u], add=True)

            fire(0, 0)

            def pair(i, carry):
                b0 = 2 * i

                @pl.when(b0 + 1 < NB)
                def _():
                    fire(b0 + 1, 1)

                process(b0, 0)

                @pl.when(b0 + 2 < NB)
                def _():
                    fire(b0 + 2, 0)

                @pl.when(b0 + 1 < NB)
                def _():
                    process(b0 + 1, 1)

                return carry

            lax.fori_loop(0, (NB + 1) // 2, pair, 0)
            pltpu.make_async_copy(gbuf[0], acc.at[dstb[0]], semS[0]).wait()
            pltpu.make_async_copy(gbuf[1], acc.at[dstb[1]], semS[1]).wait()
            plsc.subcore_barrier()

            def wb(b):
                r0 = b * CH
                pltpu.sync_copy(acc.at[pl.ds(r0, CH)],
                                out_hbm.at[pl.ds(hN + r0, CH)])

            _owned_blocks(s, wb)
            plsc.subcore_barrier()

    return k(h0f, ex, src, dst)


def _sc_msg1(h1, ex1, src, dst):
    B = 40
    EPT = E // (NC * NS)
    NB = EPT // B

    @functools.partial(
        pl.kernel,
        out_type=(
            jax.ShapeDtypeStruct((N, D), jnp.float32),
            jax.ShapeDtypeStruct((N, D), jnp.float32),
        ),
        mesh=plsc.VectorSubcoreMesh(**_MESH),
        scratch_types=(
            (pltpu.VMEM((B,), jnp.int32),) * 2,
            (pltpu.VMEM((B,), jnp.int32),) * 2,
            (pltpu.VMEM((B * HP,), jnp.float32),) * 2,
            (pltpu.VMEM((B, D), jnp.float32),) * 2,
            pltpu.VMEM((ZB, D), jnp.float32),
            pltpu.VMEM_SHARED((N, D), jnp.float32),
            pltpu.SemaphoreType.DMA,
            (pltpu.SemaphoreType.DMA,) * 2,
        ),
    )
    def k(h1_hbm, ex_hbm, src_hbm, dst_hbm, p0_hbm, p1_hbm,
          srcb, dstb, ab, gbuf, zb, acc, semM, semG):
        c = lax.axis_index("c")
        s = lax.axis_index("s")
        zero = jnp.zeros((16,), jnp.float32)
        zi = lax.iota(jnp.int32, 16) * 0

        def zrow(i, carry):
            for q in range(D // 16):
                zb[i, pl.ds(q * 16, 16)] = zero
            return carry

        lax.fori_loop(0, ZB, zrow, 0)
        _zero_blocks(s, zb, acc)
        plsc.subcore_barrier()

        base = c * (E // NC) + s * EPT

        def fire(b, u):
            eb = base + b * B
            c1 = pltpu.async_copy(src_hbm.at[pl.ds(eb, B)], srcb[u], semM)
            c2 = pltpu.async_copy(dst_hbm.at[pl.ds(eb, B)], dstb[u], semM)
            c3 = pltpu.async_copy(ex_hbm.at[pl.ds(eb * HP, B * HP)], ab[u], semM)
            c1.wait()
            c2.wait()
            c3.wait()
            pltpu.async_copy(h1_hbm.at[srcb[u]], gbuf[u], semG[u])

        def process(u):
            pltpu.make_async_copy(h1_hbm.at[srcb[u]], gbuf[u], semG[u]).wait()

            def row(r, carry2):
                av = ab[u][pl.ds(r * HP, 16)]
                a = av.at[zi].get(mode="promise_in_bounds")
                for q in range(D // 16):
                    gbuf[u][r, pl.ds(q * 16, 16)] = (
                        gbuf[u][r, pl.ds(q * 16, 16)] * a)
                return carry2

            lax.fori_loop(0, B, row, 0)
            pltpu.sync_copy(gbuf[u], acc.at[dstb[u]], add=True)

        fire(0, 0)

        def pair(i, carry):
            b0 = 2 * i
            fire(b0 + 1, 1)
            process(0)
            fire(b0 + 2, 0)
            process(1)
            return carry

        lax.fori_loop(0, NB // 2, pair, 0)
        process(0)
        plsc.subcore_barrier()

        def wb(b):
            r0 = b * CH

            @pl.when(c == 0)
            def _():
                pltpu.sync_copy(acc.at[pl.ds(r0, CH)], p0_hbm.at[pl.ds(r0, CH)])

            @pl.when(c == 1)
            def _():
                pltpu.sync_copy(acc.at[pl.ds(r0, CH)], p1_hbm.at[pl.ds(r0, CH)])

        _owned_blocks(s, wb)

    return k(h1, ex1, src, dst)


def _merge_body(p0_ref, p1_ref, dq0_ref, dq1_ref, o_ref):
    den = dq0_ref[:, 0:1] + dq1_ref[:, 0:1] + 1e-16
    o_ref[...] = (p0_ref[...] + p1_ref[...]) / den


def _merge(p0, p1, dq0, dq1):
    return pl.pallas_call(
        _merge_body,
        grid=(N // RB,),
        in_specs=[
            pl.BlockSpec((RB, D), lambda i: (i, 0)),
            pl.BlockSpec((RB, D), lambda i: (i, 0)),
            pl.BlockSpec((RB, D), lambda i: (i, 0)),
            pl.BlockSpec((RB, D), lambda i: (i, 0)),
        ],
        out_specs=pl.BlockSpec((RB, D), lambda i: (i, 0)),
        out_shape=jax.ShapeDtypeStruct((N, D), jnp.float32),
    )(p0, p1, dq0, dq1)


_BE = 8000


def _amean_body(a_ref, o_ref):
    a = a_ref[...]
    o_ref[...] = jnp.sum(a[:, :H], axis=1, keepdims=True) * (1.0 / H)


def _amean(alpha):
    return pl.pallas_call(
        _amean_body,
        grid=(E // _BE,),
        in_specs=[pl.BlockSpec((_BE, HP), lambda i: (i, 0))],
        out_specs=pl.BlockSpec((_BE, 1), lambda i: (i, 0)),
        out_shape=jax.ShapeDtypeStruct((E, 1), jnp.float32),
    )(alpha)


def kernel(reg_info, inputs, edge_index, W0, a_src0, a_dst0, W1, a_src1, a_dst1):
    x = inputs[0]
    src = edge_index[0]
    dst = edge_index[1]

    onehot = jnp.eye(HP, dtype=jnp.float32)[:H]
    Ap_src = (a_src0[:, :, None] * onehot[:, None, :]).reshape(H * D, HP)
    Ap_dst = (a_dst0[:, :, None] * onehot[:, None, :]).reshape(H * D, HP)
    Apad = jnp.concatenate(
        [Ap_src, Ap_dst, jnp.zeros((H * D, D - 2 * HP), jnp.float32)], axis=1)
    A1pad = jnp.concatenate(
        [jnp.broadcast_to(a_src1.reshape(D, 1), (D, HP)),
         jnp.broadcast_to(a_dst1.reshape(D, 1), (D, HP)),
         jnp.zeros((D, D - 2 * HP), jnp.float32)], axis=1)

    h0, asad0 = _dense0(x, W0, Apad)
    h0f = h0.reshape(H * N, D)

    ex0, dp0, dp1 = _sc_logits(asad0, src, dst)
    out0f = _sc_msg0(h0f, ex0, src, dst)
    out0 = out0f.reshape(H, N, D)

    rec0 = _recip(dp0, dp1)
    alpha0 = _sc_alpha(ex0, rec0, dst)

    h1, asad1 = _dense1(out0, dp0, dp1, W1, A1pad)
    ex1, dq0, dq1 = _sc_logits(asad1, src, dst)
    p0, p1 = _sc_msg1(h1, ex1, src, dst)

    out = _merge(p0, p1, dq0, dq1)
    anorm = _amean(alpha0.reshape(E, HP))
    return out.reshape(1, N, D), anorm.reshape(E)


# speedup vs baseline: 17.8401x; 1.0224x over previous
import functools

import jax
import jax.numpy as jnp
from jax import lax
from jax.experimental import pallas as pl
from jax.experimental.pallas import tpu as pltpu
from jax.experimental.pallas import tpu_sc as plsc

N = 10000
E = 160000
D = 128
H = 8
HP = 16
NC = 2
NS = 16
CH = 200
NBLK = N // CH
NBI = (NBLK + NS - 1) // NS
ZB = 40
RB = 1000

_MESH = dict(core_axis_name="c", subcore_axis_name="s", num_cores=NC,
             num_subcores=NS)


def _owned_blocks(s, body):
    def it(i, carry):
        b = i * NS + s

        @pl.when(b < NBLK)
        def _():
            body(b)

        return carry

    lax.fori_loop(0, NBI, it, 0)


def _zero_blocks(s, zb, acc):
    def z(b):
        for k in range(CH // ZB):
            pltpu.sync_copy(zb, acc.at[pl.ds(b * CH + k * ZB, ZB)])

    _owned_blocks(s, z)


def _dense0_body(x_ref, w_ref, apad_ref, h0_ref, asad_ref):
    xb = x_ref[...]
    hs = []
    for h in range(H):
        ph = jnp.dot(xb, w_ref[:, h * D:(h + 1) * D],
                     preferred_element_type=jnp.float32)
        h0_ref[h] = ph
        hs.append(ph)
    hb = jnp.concatenate(hs, axis=1)
    asad_ref[...] = jnp.dot(hb, apad_ref[...], preferred_element_type=jnp.float32)


def _dense0(x, W0, Apad):
    return pl.pallas_call(
        _dense0_body,
        grid=(N // RB,),
        in_specs=[
            pl.BlockSpec((RB, D), lambda i: (i, 0)),
            pl.BlockSpec((D, H * D), lambda i: (0, 0)),
            pl.BlockSpec((H * D, D), lambda i: (0, 0)),
        ],
        out_specs=[
            pl.BlockSpec((H, RB, D), lambda i: (0, i, 0)),
            pl.BlockSpec((RB, D), lambda i: (i, 0)),
        ],
        out_shape=[
            jax.ShapeDtypeStruct((H, N, D), jnp.float32),
            jax.ShapeDtypeStruct((N, D), jnp.float32),
        ],
    )(x, W0, Apad)


def _dense1_body(o_ref, dp0_ref, dp1_ref, w1_ref, a1pad_ref,
                 h1_ref, asad_ref):
    acc = jnp.zeros((RB, D), jnp.float32)
    for h in range(H):
        den = dp0_ref[:, h:h + 1] + dp1_ref[:, h:h + 1] + 1e-16
        xh = o_ref[h] / den
        xh = jnp.where(xh > 0, xh, jnp.exp(xh) - 1.0)
        acc = acc + jnp.dot(xh, w1_ref[h * D:(h + 1) * D, :],
                            preferred_element_type=jnp.float32)
    h1_ref[...] = acc
    asad_ref[...] = jnp.dot(acc, a1pad_ref[...], preferred_element_type=jnp.float32)


def _dense1(out0, dp0, dp1, W1, A1pad):
    return pl.pallas_call(
        _dense1_body,
        grid=(N // RB,),
        in_specs=[
            pl.BlockSpec((H, RB, D), lambda i: (0, i, 0)),
            pl.BlockSpec((RB, D), lambda i: (i, 0)),
            pl.BlockSpec((RB, D), lambda i: (i, 0)),
            pl.BlockSpec((H * D, D), lambda i: (0, 0)),
            pl.BlockSpec((D, D), lambda i: (0, 0)),
        ],
        out_specs=[
            pl.BlockSpec((RB, D), lambda i: (i, 0)),
            pl.BlockSpec((RB, D), lambda i: (i, 0)),
        ],
        out_shape=[
            jax.ShapeDtypeStruct((N, D), jnp.float32),
            jax.ShapeDtypeStruct((N, D), jnp.float32),
        ],
    )(out0, dp0, dp1, W1, A1pad)


def _sc_logits(asad, src, dst):
    B = 40
    EPT = E // (NC * NS)
    NB = EPT // B
    assert NB % 2 == 1

    @functools.partial(
        pl.kernel,
        out_type=(
            jax.ShapeDtypeStruct((E * HP,), jnp.float32),
            jax.ShapeDtypeStruct((N, D), jnp.float32),
            jax.ShapeDtypeStruct((N, D), jnp.float32),
        ),
        mesh=plsc.VectorSubcoreMesh(**_MESH),
        scratch_types=(
            (pltpu.VMEM((B,), jnp.int32),) * 2,
            (pltpu.VMEM((B,), jnp.int32),) * 2,
            (pltpu.VMEM((B, D), jnp.float32),) * 2,
            (pltpu.VMEM((B, D), jnp.float32),) * 2,
            pltpu.VMEM((B, D), jnp.float32),
            pltpu.VMEM((B * HP,), jnp.float32),
            pltpu.VMEM((ZB, D), jnp.float32),
            pltpu.VMEM_SHARED((N, D), jnp.float32),
            pltpu.SemaphoreType.DMA,
            (pltpu.SemaphoreType.DMA,) * 2,
        ),
    )
    def k(asad_hbm, src_hbm, dst_hbm, ex_hbm, dp0_hbm, dp1_hbm,
          srcb, dstb, gA, gB, exb, exs, zb, acc, semM, semG):
        c = lax.axis_index("c")
        s = lax.axis_index("s")
        zero = jnp.zeros((16,), jnp.float32)

        def zrow(i, carry):
            for q in range(D // 16):
                zb[i, pl.ds(q * 16, 16)] = zero
            return carry

        lax.fori_loop(0, ZB, zrow, 0)
        _zero_blocks(s, zb, acc)

        def zrow2(i, carry):
            for q in range(D // 16):
                exb[i, pl.ds(q * 16, 16)] = zero
            return carry

        lax.fori_loop(0, B, zrow2, 0)
        plsc.subcore_barrier()

        base = c * (E // NC) + s * EPT

        def fire(b, u):
            eb = base + b * B
            c1 = pltpu.async_copy(src_hbm.at[pl.ds(eb, B)], srcb[u], semM)
            c2 = pltpu.async_copy(dst_hbm.at[pl.ds(eb, B)], dstb[u], semM)
            c1.wait()
            c2.wait()
            pltpu.async_copy(asad_hbm.at[srcb[u]], gA[u], semG[u])
            pltpu.async_copy(asad_hbm.at[dstb[u]], gB[u], semG[u])

        def process(b, u):
            eb = base + b * B
            pltpu.make_async_copy(asad_hbm.at[srcb[u]], gA[u], semG[u]).wait()
            pltpu.make_async_copy(asad_hbm.at[dstb[u]], gB[u], semG[u]).wait()
            for r in range(B):
                e = gA[u][r, pl.ds(0, 16)] + gB[u][r, pl.ds(16, 16)]
                e = jnp.where(e > 0.0, e, 0.2 * e)
                ex = jnp.exp(e)
                exb[r, pl.ds(0, 16)] = ex
                exs[pl.ds(r * HP, 16)] = ex
            pltpu.sync_copy(exs, ex_hbm.at[pl.ds(eb * HP, B * HP)])
            pltpu.sync_copy(exb, acc.at[dstb[u]], add=True)

        fire(0, 0)

        def pair(i, carry):
            b0 = 2 * i
            fire(b0 + 1, 1)
            process(b0, 0)
            fire(b0 + 2, 0)
            process(b0 + 1, 1)
            return carry

        lax.fori_loop(0, NB // 2, pair, 0)
        process(NB - 1, 0)
        plsc.subcore_barrier()

        def wb(b):
            r0 = b * CH

            @pl.when(c == 0)
            def _():
                pltpu.sync_copy(acc.at[pl.ds(r0, CH)], dp0_hbm.at[pl.ds(r0, CH)])

            @pl.when(c == 1)
            def _():
                pltpu.sync_copy(acc.at[pl.ds(r0, CH)], dp1_hbm.at[pl.ds(r0, CH)])

        _owned_blocks(s, wb)

    return k(asad, src, dst)


def _recip_body(dp0_ref, dp1_ref, o_ref):
    o_ref[...] = 1.0 / (dp0_ref[...] + dp1_ref[...] + 1e-16)


def _recip(dp0, dp1):
    return pl.pallas_call(
        _recip_body,
        grid=(N // RB,),
        in_specs=[
            pl.BlockSpec((RB, D), lambda i: (i, 0)),
            pl.BlockSpec((RB, D), lambda i: (i, 0)),
        ],
        out_specs=pl.BlockSpec((RB, D), lambda i: (i, 0)),
        out_shape=jax.ShapeDtypeStruct((N, D), jnp.float32),
    )(dp0, dp1)


def _sc_alpha(ex, rec, dst):
    B = 40
    EPT = E // (NC * NS)
    NB = EPT // B

    assert NB % 2 == 1

    @functools.partial(
        pl.kernel,
        out_type=jax.ShapeDtypeStruct((E * HP,), jnp.float32),
        mesh=plsc.VectorSubcoreMesh(**_MESH),
        scratch_types=(
            (pltpu.VMEM((B,), jnp.int32),) * 2,
            (pltpu.VMEM((B * HP,), jnp.float32),) * 2,
            (pltpu.VMEM((B, D), jnp.float32),) * 2,
            pltpu.VMEM((B * HP,), jnp.float32),
            pltpu.SemaphoreType.DMA,
            (pltpu.SemaphoreType.DMA,) * 2,
        ),
    )
    def k(ex_hbm, rec_hbm, dst_hbm, al_hbm, dstb, exs, grec, alb, semM, semG):
        c = lax.axis_index("c")
        s = lax.axis_index("s")
        base = c * (E // NC) + s * EPT

        def fire(b, u):
            eb = base + b * B
            c1 = pltpu.async_copy(dst_hbm.at[pl.ds(eb, B)], dstb[u], semM)
            c2 = pltpu.async_copy(ex_hbm.at[pl.ds(eb * HP, B * HP)], exs[u], semM)
            c1.wait()
            c2.wait()
            pltpu.async_copy(rec_hbm.at[dstb[u]], grec[u], semG[u])

        def process(b, u):
            eb = base + b * B
            pltpu.make_async_copy(rec_hbm.at[dstb[u]], grec[u], semG[u]).wait()
            for r in range(B):
                alb[pl.ds(r * HP, 16)] = (exs[u][pl.ds(r * HP, 16)]
                                          * grec[u][r, pl.ds(0, 16)])
            pltpu.sync_copy(alb, al_hbm.at[pl.ds(eb * HP, B * HP)])

        fire(0, 0)

        def pair(i, carry):
            b0 = 2 * i
            fire(b0 + 1, 1)
            process(b0, 0)
            fire(b0 + 2, 0)
            process(b0 + 1, 1)
            return carry

        lax.fori_loop(0, NB // 2, pair, 0)
        process(NB - 1, 0)

    return k(ex, rec, dst)


def _sc_msg0(h0f, ex, src, dst):
    B = 80
    G = 5
    MB = G * B
    EPT = E // NS
    NB = EPT // B
    HPC = H // NC

    assert NB % 2 == 1 and B % 16 == 0 and NB % G == 0

    @functools.partial(
        pl.kernel,
        out_type=jax.ShapeDtypeStruct((H * N, D), jnp.float32),
        mesh=plsc.VectorSubcoreMesh(**_MESH),
        scratch_types=(
            pltpu.VMEM((MB,), jnp.int32),
            pltpu.VMEM((MB,), jnp.int32),
            (pltpu.VMEM((B * HP,), jnp.float32),) * 2,
            (pltpu.VMEM((B,), jnp.int32),) * 2,
            (pltpu.VMEM((B,), jnp.int32),) * 2,
            (pltpu.VMEM((B, D), jnp.float32),) * 2,
            pltpu.VMEM((ZB, D), jnp.float32),
            pltpu.VMEM_SHARED((N, D), jnp.float32),
            pltpu.SemaphoreType.DMA,
            (pltpu.SemaphoreType.DMA,) * 2,
            (pltpu.SemaphoreType.DMA,) * 2,
        ),
    )
    def k(h0_hbm, ex_hbm, src_hbm, dst_hbm, out_hbm,
          srcm, dstm, ab, srcb2, dstb, gbuf, zb, acc, semM, semG, semS):
        c = lax.axis_index("c")
        s = lax.axis_index("s")
        zero = jnp.zeros((16,), jnp.float32)
        iota = lax.iota(jnp.int32, 16)
        zi = iota * 0

        def zrow(i, carry):
            for q in range(D // 16):
                zb[i, pl.ds(q * 16, 16)] = zero
            return carry

        lax.fori_loop(0, ZB, zrow, 0)

        for j in range(HPC):
            h = c * HPC + j
            hN = h * N
            hvec = jnp.full((16,), h, jnp.int32)
            _zero_blocks(s, zb, acc)
            plsc.subcore_barrier()

            def fire(b, u):
                kk = lax.rem(b, G)

                @pl.when(kk == 0)
                def _():
                    eb = s * EPT + b * B
                    c1 = pltpu.async_copy(src_hbm.at[pl.ds(eb, MB)], srcm, semM)
                    c2 = pltpu.async_copy(dst_hbm.at[pl.ds(eb, MB)], dstm, semM)
                    c1.wait()
                    c2.wait()

                @pl.when(b >= 2)
                def _():
                    pltpu.make_async_copy(gbuf[u], acc.at[dstb[u]],
                                          semS[u]).wait()

                koff = kk * B
                for q in range(B // 16):
                    srcb2[u][pl.ds(q * 16, 16)] = (
                        srcm[pl.ds(koff + q * 16, 16)] + hN)
                    dstb[u][pl.ds(q * 16, 16)] = dstm[pl.ds(koff + q * 16, 16)]
                eb2 = s * EPT + b * B
                pltpu.async_copy(ex_hbm.at[pl.ds(eb2 * HP, B * HP)], ab[u],
                                 semG[u])
                pltpu.async_copy(h0_hbm.at[srcb2[u]], gbuf[u], semG[u])

            def process(b, u):
                eb2 = s * EPT + b * B
                pltpu.make_async_copy(ex_hbm.at[pl.ds(eb2 * HP, B * HP)],
                                      ab[u], semG[u]).wait()
                pltpu.make_async_copy(h0_hbm.at[srcb2[u]], gbuf[u],
                                      semG[u]).wait()

                def grp(rg, carry2):
                    rbase = rg * 16
                    for r16 in range(16):
                        av = ab[u][pl.ds((rbase + r16) * HP, 16)]
                        a = av.at[hvec].get(mode="promise_in_bounds")
                        for q in range(D // 16):
                            gbuf[u][rbase + r16, pl.ds(q * 16, 16)] = (
                                gbuf[u][rbase + r16, pl.ds(q * 16, 16)] * a)
                    return carry2

                lax.fori_loop(0, B // 16, grp, 0)
                pltpu.async_copy(gbuf[u], acc.at[dstb[u]], semS[u], add=True)

            fire(0, 0)

            def pair(i, carry):
                b0 = 2 * i

                @pl.when(b0 + 1 < NB)
                def _():
                    fire(b0 + 1, 1)

                process(b0, 0)

                @pl.when(b0 + 2 < NB)
                def _():
                    fire(b0 + 2, 0)

                @pl.when(b0 + 1 < NB)
                def _():
                    process(b0 + 1, 1)

                return carry

            lax.fori_loop(0, (NB + 1) // 2, pair, 0)
            pltpu.make_async_copy(gbuf[0], acc.at[dstb[0]], semS[0]).wait()
            pltpu.make_async_copy(gbuf[1], acc.at[dstb[1]], semS[1]).wait()
            plsc.subcore_barrier()

            def wb(b):
                r0 = b * CH
                pltpu.sync_copy(acc.at[pl.ds(r0, CH)],
                                out_hbm.at[pl.ds(hN + r0, CH)])

            _owned_blocks(s, wb)
            plsc.subcore_barrier()

    return k(h0f, ex, src, dst)


def _sc_msg1(h1, ex1, src, dst):
    B = 40
    EPT = E // (NC * NS)
    NB = EPT // B

    @functools.partial(
        pl.kernel,
        out_type=(
            jax.ShapeDtypeStruct((N, D), jnp.float32),
            jax.ShapeDtypeStruct((N, D), jnp.float32),
        ),
        mesh=plsc.VectorSubcoreMesh(**_MESH),
        scratch_types=(
            (pltpu.VMEM((B,), jnp.int32),) * 2,
            (pltpu.VMEM((B,), jnp.int32),) * 2,
            (pltpu.VMEM((B * HP,), jnp.float32),) * 2,
            (pltpu.VMEM((B, D), jnp.float32),) * 2,
            pltpu.VMEM((ZB, D), jnp.float32),
            pltpu.VMEM_SHARED((N, D), jnp.float32),
            pltpu.SemaphoreType.DMA,
            (pltpu.SemaphoreType.DMA,) * 2,
        ),
    )
    def k(h1_hbm, ex_hbm, src_hbm, dst_hbm, p0_hbm, p1_hbm,
          srcb, dstb, ab, gbuf, zb, acc, semM, semG):
        c = lax.axis_index("c")
        s = lax.axis_index("s")
        zero = jnp.zeros((16,), jnp.float32)
        zi = lax.iota(jnp.int32, 16) * 0

        def zrow(i, carry):
            for q in range(D // 16):
                zb[i, pl.ds(q * 16, 16)] = zero
            return carry

        lax.fori_loop(0, ZB, zrow, 0)
        _zero_blocks(s, zb, acc)
        plsc.subcore_barrier()

        base = c * (E // NC) + s * EPT

        def fire(b, u):
            eb = base + b * B
            c1 = pltpu.async_copy(src_hbm.at[pl.ds(eb, B)], srcb[u], semM)
            c2 = pltpu.async_copy(dst_hbm.at[pl.ds(eb, B)], dstb[u], semM)
            c3 = pltpu.async_copy(ex_hbm.at[pl.ds(eb * HP, B * HP)], ab[u], semM)
            c1.wait()
            c2.wait()
            c3.wait()
            pltpu.async_copy(h1_hbm.at[srcb[u]], gbuf[u], semG[u])

        def process(u):
            pltpu.make_async_copy(h1_hbm.at[srcb[u]], gbuf[u], semG[u]).wait()

            def row(r, carry2):
                av = ab[u][pl.ds(r * HP, 16)]
                a = av.at[zi].get(mode="promise_in_bounds")
                for q in range(D // 16):
                    gbuf[u][r, pl.ds(q * 16, 16)] = (
                        gbuf[u][r, pl.ds(q * 16, 16)] * a)
                return carry2

            lax.fori_loop(0, B, row, 0)
            pltpu.sync_copy(gbuf[u], acc.at[dstb[u]], add=True)

        fire(0, 0)

        def pair(i, carry):
            b0 = 2 * i
            fire(b0 + 1, 1)
            process(0)
            fire(b0 + 2, 0)
            process(1)
            return carry

        lax.fori_loop(0, NB // 2, pair, 0)
        process(0)
        plsc.subcore_barrier()

        def wb(b):
            r0 = b * CH

            @pl.when(c == 0)
            def _():
                pltpu.sync_copy(acc.at[pl.ds(r0, CH)], p0_hbm.at[pl.ds(r0, CH)])

            @pl.when(c == 1)
            def _():
                pltpu.sync_copy(acc.at[pl.ds(r0, CH)], p1_hbm.at[pl.ds(r0, CH)])

        _owned_blocks(s, wb)

    return k(h1, ex1, src, dst)


def _merge_body(p0_ref, p1_ref, dq0_ref, dq1_ref, o_ref):
    den = dq0_ref[:, 0:1] + dq1_ref[:, 0:1] + 1e-16
    o_ref[...] = (p0_ref[...] + p1_ref[...]) / den


def _merge(p0, p1, dq0, dq1):
    return pl.pallas_call(
        _merge_body,
        grid=(N // RB,),
        in_specs=[
            pl.BlockSpec((RB, D), lambda i: (i, 0)),
            pl.BlockSpec((RB, D), lambda i: (i, 0)),
            pl.BlockSpec((RB, D), lambda i: (i, 0)),
            pl.BlockSpec((RB, D), lambda i: (i, 0)),
        ],
        out_specs=pl.BlockSpec((RB, D), lambda i: (i, 0)),
        out_shape=jax.ShapeDtypeStruct((N, D), jnp.float32),
    )(p0, p1, dq0, dq1)


_BE = 8000


def _amean_body(a_ref, o_ref):
    a = a_ref[...]
    o_ref[...] = jnp.sum(a[:, :H], axis=1, keepdims=True) * (1.0 / H)


def _amean(alpha):
    return pl.pallas_call(
        _amean_body,
        grid=(E // _BE,),
        in_specs=[pl.BlockSpec((_BE, HP), lambda i: (i, 0))],
        out_specs=pl.BlockSpec((_BE, 1), lambda i: (i, 0)),
        out_shape=jax.ShapeDtypeStruct((E, 1), jnp.float32),
    )(alpha)


def kernel(reg_info, inputs, edge_index, W0, a_src0, a_dst0, W1, a_src1, a_dst1):
    x = inputs[0]
    src = edge_index[0]
    dst = edge_index[1]

    onehot = jnp.eye(HP, dtype=jnp.float32)[:H]
    Ap_src = (a_src0[:, :, None] * onehot[:, None, :]).reshape(H * D, HP)
    Ap_dst = (a_dst0[:, :, None] * onehot[:, None, :]).reshape(H * D, HP)
    Apad = jnp.concatenate(
        [Ap_src, Ap_dst, jnp.zeros((H * D, D - 2 * HP), jnp.float32)], axis=1)
    A1pad = jnp.concatenate(
        [jnp.broadcast_to(a_src1.reshape(D, 1), (D, HP)),
         jnp.broadcast_to(a_dst1.reshape(D, 1), (D, HP)),
         jnp.zeros((D, D - 2 * HP), jnp.float32)], axis=1)

    h0, asad0 = _dense0(x, W0, Apad)
    h0f = h0.reshape(H * N, D)

    ex0, dp0, dp1 = _sc_logits(asad0, src, dst)
    out0f = _sc_msg0(h0f, ex0, src, dst)
    out0 = out0f.reshape(H, N, D)

    rec0 = _recip(dp0, dp1)
    alpha0 = _sc_alpha(ex0, rec0, dst)

    h1, asad1 = _dense1(out0, dp0, dp1, W1, A1pad)
    ex1, dq0, dq1 = _sc_logits(asad1, src, dst)
    p0, p1 = _sc_msg1(h1, ex1, src, dst)

    out = _merge(p0, p1, dq0, dq1)
    anorm = _amean(alpha0.reshape(E, HP))
    return out.reshape(1, N, D), anorm.reshape(E)


# speedup vs baseline: 17.9755x; 1.0076x over previous
import functools

import jax
import jax.numpy as jnp
from jax import lax
from jax.experimental import pallas as pl
from jax.experimental.pallas import tpu as pltpu
from jax.experimental.pallas import tpu_sc as plsc

N = 10000
E = 160000
D = 128
H = 8
HP = 16
NC = 2
NS = 16
CH = 200
NBLK = N // CH
NBI = (NBLK + NS - 1) // NS
ZB = 40
RB = 1000

_MESH = dict(core_axis_name="c", subcore_axis_name="s", num_cores=NC,
             num_subcores=NS)


def _owned_blocks(s, body):
    def it(i, carry):
        b = i * NS + s

        @pl.when(b < NBLK)
        def _():
            body(b)

        return carry

    lax.fori_loop(0, NBI, it, 0)


def _zero_blocks(s, zb, acc):
    def z(b):
        for k in range(CH // ZB):
            pltpu.sync_copy(zb, acc.at[pl.ds(b * CH + k * ZB, ZB)])

    _owned_blocks(s, z)


def _dense0_body(x_ref, w_ref, apad_ref, h0_ref, asad_ref):
    xb = x_ref[...]
    hs = []
    for h in range(H):
        ph = jnp.dot(xb, w_ref[:, h * D:(h + 1) * D],
                     preferred_element_type=jnp.float32)
        h0_ref[h] = ph
        hs.append(ph)
    hb = jnp.concatenate(hs, axis=1)
    asad_ref[...] = jnp.dot(hb, apad_ref[...], preferred_element_type=jnp.float32)


def _dense0(x, W0, Apad):
    return pl.pallas_call(
        _dense0_body,
        grid=(N // RB,),
        in_specs=[
            pl.BlockSpec((RB, D), lambda i: (i, 0)),
            pl.BlockSpec((D, H * D), lambda i: (0, 0)),
            pl.BlockSpec((H * D, D), lambda i: (0, 0)),
        ],
        out_specs=[
            pl.BlockSpec((H, RB, D), lambda i: (0, i, 0)),
            pl.BlockSpec((RB, D), lambda i: (i, 0)),
        ],
        out_shape=[
            jax.ShapeDtypeStruct((H, N, D), jnp.float32),
            jax.ShapeDtypeStruct((N, D), jnp.float32),
        ],
    )(x, W0, Apad)


def _dense1_body(o_ref, dp0_ref, dp1_ref, w1_ref, a1pad_ref,
                 h1_ref, asad_ref):
    acc = jnp.zeros((RB, D), jnp.float32)
    for h in range(H):
        den = dp0_ref[:, h:h + 1] + dp1_ref[:, h:h + 1] + 1e-16
        xh = o_ref[h] / den
        xh = jnp.where(xh > 0, xh, jnp.exp(xh) - 1.0)
        acc = acc + jnp.dot(xh, w1_ref[h * D:(h + 1) * D, :],
                            preferred_element_type=jnp.float32)
    h1_ref[...] = acc
    asad_ref[...] = jnp.dot(acc, a1pad_ref[...], preferred_element_type=jnp.float32)


def _dense1(out0, dp0, dp1, W1, A1pad):
    return pl.pallas_call(
        _dense1_body,
        grid=(N // RB,),
        in_specs=[
            pl.BlockSpec((H, RB, D), lambda i: (0, i, 0)),
            pl.BlockSpec((RB, D), lambda i: (i, 0)),
            pl.BlockSpec((RB, D), lambda i: (i, 0)),
            pl.BlockSpec((H * D, D), lambda i: (0, 0)),
            pl.BlockSpec((D, D), lambda i: (0, 0)),
        ],
        out_specs=[
            pl.BlockSpec((RB, D), lambda i: (i, 0)),
            pl.BlockSpec((RB, D), lambda i: (i, 0)),
        ],
        out_shape=[
            jax.ShapeDtypeStruct((N, D), jnp.float32),
            jax.ShapeDtypeStruct((N, D), jnp.float32),
        ],
    )(out0, dp0, dp1, W1, A1pad)


def _sc_logits(asad, src, dst):
    B = 40
    EPT = E // (NC * NS)
    NB = EPT // B
    assert NB % 2 == 1

    @functools.partial(
        pl.kernel,
        out_type=(
            jax.ShapeDtypeStruct((E * HP,), jnp.float32),
            jax.ShapeDtypeStruct((N, D), jnp.float32),
            jax.ShapeDtypeStruct((N, D), jnp.float32),
        ),
        mesh=plsc.VectorSubcoreMesh(**_MESH),
        scratch_types=(
            (pltpu.VMEM((B,), jnp.int32),) * 2,
            (pltpu.VMEM((B,), jnp.int32),) * 2,
            (pltpu.VMEM((B, D), jnp.float32),) * 2,
            (pltpu.VMEM((B, D), jnp.float32),) * 2,
            (pltpu.VMEM((B, D), jnp.float32),) * 2,
            (pltpu.VMEM((B * HP,), jnp.float32),) * 2,
            pltpu.VMEM((ZB, D), jnp.float32),
            pltpu.VMEM_SHARED((N, D), jnp.float32),
            pltpu.SemaphoreType.DMA,
            (pltpu.SemaphoreType.DMA,) * 2,
            (pltpu.SemaphoreType.DMA,) * 2,
            (pltpu.SemaphoreType.DMA,) * 2,
        ),
    )
    def k(asad_hbm, src_hbm, dst_hbm, ex_hbm, dp0_hbm, dp1_hbm,
          srcb, dstb, gA, gB, exb, exs, zb, acc, semM, semG, semS, semW):
        c = lax.axis_index("c")
        s = lax.axis_index("s")
        zero = jnp.zeros((16,), jnp.float32)

        def zrow(i, carry):
            for q in range(D // 16):
                zb[i, pl.ds(q * 16, 16)] = zero
            return carry

        lax.fori_loop(0, ZB, zrow, 0)
        _zero_blocks(s, zb, acc)

        def zrow2(i, carry):
            for q in range(D // 16):
                exb[0][i, pl.ds(q * 16, 16)] = zero
                exb[1][i, pl.ds(q * 16, 16)] = zero
            return carry

        lax.fori_loop(0, B, zrow2, 0)
        plsc.subcore_barrier()

        base = c * (E // NC) + s * EPT

        def fire(b, u):
            eb = base + b * B

            @pl.when(b >= 2)
            def _():
                pltpu.make_async_copy(
                    exs[u], ex_hbm.at[pl.ds(eb * HP, B * HP)], semW[u]).wait()
                pltpu.make_async_copy(exb[u], acc.at[dstb[u]], semS[u]).wait()

            c1 = pltpu.async_copy(src_hbm.at[pl.ds(eb, B)], srcb[u], semM)
            c2 = pltpu.async_copy(dst_hbm.at[pl.ds(eb, B)], dstb[u], semM)
            c1.wait()
            c2.wait()
            pltpu.async_copy(asad_hbm.at[srcb[u]], gA[u], semG[u])
            pltpu.async_copy(asad_hbm.at[dstb[u]], gB[u], semG[u])

        def process(b, u):
            eb = base + b * B
            pltpu.make_async_copy(asad_hbm.at[srcb[u]], gA[u], semG[u]).wait()
            pltpu.make_async_copy(asad_hbm.at[dstb[u]], gB[u], semG[u]).wait()
            for r in range(B):
                e = gA[u][r, pl.ds(0, 16)] + gB[u][r, pl.ds(16, 16)]
                e = jnp.where(e > 0.0, e, 0.2 * e)
                ex = jnp.exp(e)
                exb[u][r, pl.ds(0, 16)] = ex
                exs[u][pl.ds(r * HP, 16)] = ex
            pltpu.async_copy(exs[u], ex_hbm.at[pl.ds(eb * HP, B * HP)], semW[u])
            pltpu.async_copy(exb[u], acc.at[dstb[u]], semS[u], add=True)

        fire(0, 0)

        def pair(i, carry):
            b0 = 2 * i
            fire(b0 + 1, 1)
            process(b0, 0)
            fire(b0 + 2, 0)
            process(b0 + 1, 1)
            return carry

        lax.fori_loop(0, NB // 2, pair, 0)
        process(NB - 1, 0)
        for u in (0, 1):
            pltpu.make_async_copy(
                exs[u], ex_hbm.at[pl.ds(0, B * HP)], semW[u]).wait()
            pltpu.make_async_copy(exb[u], acc.at[dstb[u]], semS[u]).wait()
        plsc.subcore_barrier()

        def wb(b):
            r0 = b * CH

            @pl.when(c == 0)
            def _():
                pltpu.sync_copy(acc.at[pl.ds(r0, CH)], dp0_hbm.at[pl.ds(r0, CH)])

            @pl.when(c == 1)
            def _():
                pltpu.sync_copy(acc.at[pl.ds(r0, CH)], dp1_hbm.at[pl.ds(r0, CH)])

        _owned_blocks(s, wb)

    return k(asad, src, dst)


def _recip_body(dp0_ref, dp1_ref, o_ref):
    o_ref[...] = 1.0 / (dp0_ref[...] + dp1_ref[...] + 1e-16)


def _recip(dp0, dp1):
    return pl.pallas_call(
        _recip_body,
        grid=(N // RB,),
        in_specs=[
            pl.BlockSpec((RB, D), lambda i: (i, 0)),
            pl.BlockSpec((RB, D), lambda i: (i, 0)),
        ],
        out_specs=pl.BlockSpec((RB, D), lambda i: (i, 0)),
        out_shape=jax.ShapeDtypeStruct((N, D), jnp.float32),
    )(dp0, dp1)


def _sc_alpha(ex, rec, dst):
    B = 40
    EPT = E // (NC * NS)
    NB = EPT // B

    assert NB % 2 == 1

    @functools.partial(
        pl.kernel,
        out_type=jax.ShapeDtypeStruct((E * HP,), jnp.float32),
        mesh=plsc.VectorSubcoreMesh(**_MESH),
        scratch_types=(
            (pltpu.VMEM((B,), jnp.int32),) * 2,
            (pltpu.VMEM((B * HP,), jnp.float32),) * 2,
            (pltpu.VMEM((B, D), jnp.float32),) * 2,
            (pltpu.VMEM((B * HP,), jnp.float32),) * 2,
            pltpu.SemaphoreType.DMA,
            (pltpu.SemaphoreType.DMA,) * 2,
            (pltpu.SemaphoreType.DMA,) * 2,
        ),
    )
    def k(ex_hbm, rec_hbm, dst_hbm, al_hbm, dstb, exs, grec, alb, semM, semG,
          semS):
        c = lax.axis_index("c")
        s = lax.axis_index("s")
        base = c * (E // NC) + s * EPT

        def fire(b, u):
            eb = base + b * B

            @pl.when(b >= 2)
            def _():
                pltpu.make_async_copy(
                    alb[u], al_hbm.at[pl.ds(eb * HP, B * HP)], semS[u]).wait()

            c1 = pltpu.async_copy(dst_hbm.at[pl.ds(eb, B)], dstb[u], semM)
            c2 = pltpu.async_copy(ex_hbm.at[pl.ds(eb * HP, B * HP)], exs[u], semM)
            c1.wait()
            c2.wait()
            pltpu.async_copy(rec_hbm.at[dstb[u]], grec[u], semG[u])

        def process(b, u):
            eb = base + b * B
            pltpu.make_async_copy(rec_hbm.at[dstb[u]], grec[u], semG[u]).wait()
            for r in range(B):
                alb[u][pl.ds(r * HP, 16)] = (exs[u][pl.ds(r * HP, 16)]
                                             * grec[u][r, pl.ds(0, 16)])
            pltpu.async_copy(alb[u], al_hbm.at[pl.ds(eb * HP, B * HP)], semS[u])

        fire(0, 0)

        def pair(i, carry):
            b0 = 2 * i
            fire(b0 + 1, 1)
            process(b0, 0)
            fire(b0 + 2, 0)
            process(b0 + 1, 1)
            return carry

        lax.fori_loop(0, NB // 2, pair, 0)
        process(NB - 1, 0)
        for u in (0, 1):
            pltpu.make_async_copy(
                alb[u], al_hbm.at[pl.ds(0, B * HP)], semS[u]).wait()

    return k(ex, rec, dst)


def _sc_msg0(h0f, ex, src, dst):
    B = 80
    G = 5
    MB = G * B
    EPT = E // NS
    NB = EPT // B
    HPC = H // NC

    assert NB % 2 == 1 and B % 16 == 0 and NB % G == 0

    @functools.partial(
        pl.kernel,
        out_type=jax.ShapeDtypeStruct((H * N, D), jnp.float32),
        mesh=plsc.VectorSubcoreMesh(**_MESH),
        scratch_types=(
            pltpu.VMEM((MB,), jnp.int32),
            pltpu.VMEM((MB,), jnp.int32),
            (pltpu.VMEM((B * HP,), jnp.float32),) * 2,
            (pltpu.VMEM((B,), jnp.int32),) * 2,
            (pltpu.VMEM((B,), jnp.int32),) * 2,
            (pltpu.VMEM((B, D), jnp.float32),) * 2,
            pltpu.VMEM((ZB, D), jnp.float32),
            pltpu.VMEM_SHARED((N, D), jnp.float32),
            pltpu.SemaphoreType.DMA,
            (pltpu.SemaphoreType.DMA,) * 2,
            (pltpu.SemaphoreType.DMA,) * 2,
        ),
    )
    def k(h0_hbm, ex_hbm, src_hbm, dst_hbm, out_hbm,
          srcm, dstm, ab, srcb2, dstb, gbuf, zb, acc, semM, semG, semS):
        c = lax.axis_index("c")
        s = lax.axis_index("s")
        zero = jnp.zeros((16,), jnp.float32)
        iota = lax.iota(jnp.int32, 16)
        zi = iota * 0

        def zrow(i, carry):
            for q in range(D // 16):
                zb[i, pl.ds(q * 16, 16)] = zero
            return carry

        lax.fori_loop(0, ZB, zrow, 0)

        for j in range(HPC):
            h = c * HPC + j
            hN = h * N
            hvec = jnp.full((16,), h, jnp.int32)
            _zero_blocks(s, zb, acc)
            plsc.subcore_barrier()

            def fire(b, u):
                kk = lax.rem(b, G)

                @pl.when(kk == 0)
                def _():
                    eb = s * EPT + b * B
                    c1 = pltpu.async_copy(src_hbm.at[pl.ds(eb, MB)], srcm, semM)
                    c2 = pltpu.async_copy(dst_hbm.at[pl.ds(eb, MB)], dstm, semM)
                    c1.wait()
                    c2.wait()

                @pl.when(b >= 2)
                def _():
                    pltpu.make_async_copy(gbuf[u], acc.at[dstb[u]],
                                          semS[u]).wait()

                koff = kk * B
                for q in range(B // 16):
                    srcb2[u][pl.ds(q * 16, 16)] = (
                        srcm[pl.ds(koff + q * 16, 16)] + hN)
                    dstb[u][pl.ds(q * 16, 16)] = dstm[pl.ds(koff + q * 16, 16)]
                eb2 = s * EPT + b * B
                pltpu.async_copy(ex_hbm.at[pl.ds(eb2 * HP, B * HP)], ab[u],
                                 semG[u])
                pltpu.async_copy(h0_hbm.at[srcb2[u]], gbuf[u], semG[u])

            def process(b, u):
                eb2 = s * EPT + b * B
                pltpu.make_async_copy(ex_hbm.at[pl.ds(eb2 * HP, B * HP)],
                                      ab[u], semG[u]).wait()
                pltpu.make_async_copy(h0_hbm.at[srcb2[u]], gbuf[u],
                                      semG[u]).wait()

                def grp(rg, carry2):
                    rbase = rg * 16
                    for r16 in range(16):
                        av = ab[u][pl.ds((rbase + r16) * HP, 16)]
                        a = av.at[hvec].get(mode="promise_in_bounds")
                        for q in range(D // 16):
                            gbuf[u][rbase + r16, pl.ds(q * 16, 16)] = (
                                gbuf[u][rbase + r16, pl.ds(q * 16, 16)] * a)
                    return carry2

                lax.fori_loop(0, B // 16, grp, 0)
                pltpu.async_copy(gbuf[u], acc.at[dstb[u]], semS[u], add=True)

            fire(0, 0)

            def pair(i, carry):
                b0 = 2 * i

                @pl.when(b0 + 1 < NB)
                def _():
                    fire(b0 + 1, 1)

                process(b0, 0)

                @pl.when(b0 + 2 < NB)
                def _():
                    fire(b0 + 2, 0)

                @pl.when(b0 + 1 < NB)
                def _():
                    process(b0 + 1, 1)

                return carry

            lax.fori_loop(0, (NB + 1) // 2, pair, 0)
            pltpu.make_async_copy(gbuf[0], acc.at[dstb[0]], semS[0]).wait()
            pltpu.make_async_copy(gbuf[1], acc.at[dstb[1]], semS[1]).wait()
            plsc.subcore_barrier()

            def wb(b):
                r0 = b * CH
                pltpu.sync_copy(acc.at[pl.ds(r0, CH)],
                                out_hbm.at[pl.ds(hN + r0, CH)])

            _owned_blocks(s, wb)
            plsc.subcore_barrier()

    return k(h0f, ex, src, dst)


def _sc_msg1(h1, ex1, src, dst):
    B = 40
    EPT = E // (NC * NS)
    NB = EPT // B

    @functools.partial(
        pl.kernel,
        out_type=(
            jax.ShapeDtypeStruct((N, D), jnp.float32),
            jax.ShapeDtypeStruct((N, D), jnp.float32),
        ),
        mesh=plsc.VectorSubcoreMesh(**_MESH),
        scratch_types=(
            (pltpu.VMEM((B,), jnp.int32),) * 2,
            (pltpu.VMEM((B,), jnp.int32),) * 2,
            (pltpu.VMEM((B * HP,), jnp.float32),) * 2,
            (pltpu.VMEM((B, D), jnp.float32),) * 2,
            pltpu.VMEM((ZB, D), jnp.float32),
            pltpu.VMEM_SHARED((N, D), jnp.float32),
            pltpu.SemaphoreType.DMA,
            (pltpu.SemaphoreType.DMA,) * 2,
            (pltpu.SemaphoreType.DMA,) * 2,
        ),
    )
    def k(h1_hbm, ex_hbm, src_hbm, dst_hbm, p0_hbm, p1_hbm,
          srcb, dstb, ab, gbuf, zb, acc, semM, semG, semS):
        c = lax.axis_index("c")
        s = lax.axis_index("s")
        zero = jnp.zeros((16,), jnp.float32)
        zi = lax.iota(jnp.int32, 16) * 0

        def zrow(i, carry):
            for q in range(D // 16):
                zb[i, pl.ds(q * 16, 16)] = zero
            return carry

        lax.fori_loop(0, ZB, zrow, 0)
        _zero_blocks(s, zb, acc)
        plsc.subcore_barrier()

        base = c * (E // NC) + s * EPT

        def fire(b, u):
            eb = base + b * B

            @pl.when(b >= 2)
            def _():
                pltpu.make_async_copy(gbuf[u], acc.at[dstb[u]], semS[u]).wait()

            c1 = pltpu.async_copy(src_hbm.at[pl.ds(eb, B)], srcb[u], semM)
            c2 = pltpu.async_copy(dst_hbm.at[pl.ds(eb, B)], dstb[u], semM)
            c3 = pltpu.async_copy(ex_hbm.at[pl.ds(eb * HP, B * HP)], ab[u], semM)
            c1.wait()
            c2.wait()
            c3.wait()
            pltpu.async_copy(h1_hbm.at[srcb[u]], gbuf[u], semG[u])

        def process(u):
            pltpu.make_async_copy(h1_hbm.at[srcb[u]], gbuf[u], semG[u]).wait()

            def grp(g, carry2):
                for r8 in range(8):
                    r = g * 8 + r8
                    av = ab[u][pl.ds(r * HP, 16)]
                    a = av.at[zi].get(mode="promise_in_bounds")
                    for q in range(D // 16):
                        gbuf[u][r, pl.ds(q * 16, 16)] = (
                            gbuf[u][r, pl.ds(q * 16, 16)] * a)
                return carry2

            lax.fori_loop(0, B // 8, grp, 0)
            pltpu.async_copy(gbuf[u], acc.at[dstb[u]], semS[u], add=True)

        def fireproc(i, carry):
            b0 = 2 * i
            fire(b0 + 1, 1)
            process(0)
            fire(b0 + 2, 0)
            process(1)
            return carry

        fire(0, 0)
        lax.fori_loop(0, NB // 2, fireproc, 0)
        process(0)
        pltpu.make_async_copy(gbuf[0], acc.at[dstb[0]], semS[0]).wait()
        pltpu.make_async_copy(gbuf[1], acc.at[dstb[1]], semS[1]).wait()
        plsc.subcore_barrier()

        def wb(b):
            r0 = b * CH

            @pl.when(c == 0)
            def _():
                pltpu.sync_copy(acc.at[pl.ds(r0, CH)], p0_hbm.at[pl.ds(r0, CH)])

            @pl.when(c == 1)
            def _():
                pltpu.sync_copy(acc.at[pl.ds(r0, CH)], p1_hbm.at[pl.ds(r0, CH)])

        _owned_blocks(s, wb)

    return k(h1, ex1, src, dst)


def _merge_body(p0_ref, p1_ref, dq0_ref, dq1_ref, o_ref):
    den = dq0_ref[:, 0:1] + dq1_ref[:, 0:1] + 1e-16
    o_ref[...] = (p0_ref[...] + p1_ref[...]) / den


def _merge(p0, p1, dq0, dq1):
    return pl.pallas_call(
        _merge_body,
        grid=(N // RB,),
        in_specs=[
            pl.BlockSpec((RB, D), lambda i: (i, 0)),
            pl.BlockSpec((RB, D), lambda i: (i, 0)),
            pl.BlockSpec((RB, D), lambda i: (i, 0)),
            pl.BlockSpec((RB, D), lambda i: (i, 0)),
        ],
        out_specs=pl.BlockSpec((RB, D), lambda i: (i, 0)),
        out_shape=jax.ShapeDtypeStruct((N, D), jnp.float32),
    )(p0, p1, dq0, dq1)


_BE = 8000


def _amean_body(a_ref, o_ref):
    a = a_ref[...]
    o_ref[...] = jnp.sum(a[:, :H], axis=1, keepdims=True) * (1.0 / H)


def _amean(alpha):
    return pl.pallas_call(
        _amean_body,
        grid=(E // _BE,),
        in_specs=[pl.BlockSpec((_BE, HP), lambda i: (i, 0))],
        out_specs=pl.BlockSpec((_BE, 1), lambda i: (i, 0)),
        out_shape=jax.ShapeDtypeStruct((E, 1), jnp.float32),
    )(alpha)


def kernel(reg_info, inputs, edge_index, W0, a_src0, a_dst0, W1, a_src1, a_dst1):
    x = inputs[0]
    src = edge_index[0]
    dst = edge_index[1]

    onehot = jnp.eye(HP, dtype=jnp.float32)[:H]
    Ap_src = (a_src0[:, :, None] * onehot[:, None, :]).reshape(H * D, HP)
    Ap_dst = (a_dst0[:, :, None] * onehot[:, None, :]).reshape(H * D, HP)
    Apad = jnp.concatenate(
        [Ap_src, Ap_dst, jnp.zeros((H * D, D - 2 * HP), jnp.float32)], axis=1)
    A1pad = jnp.concatenate(
        [jnp.broadcast_to(a_src1.reshape(D, 1), (D, HP)),
         jnp.broadcast_to(a_dst1.reshape(D, 1), (D, HP)),
         jnp.zeros((D, D - 2 * HP), jnp.float32)], axis=1)

    h0, asad0 = _dense0(x, W0, Apad)
    h0f = h0.reshape(H * N, D)

    ex0, dp0, dp1 = _sc_logits(asad0, src, dst)
    out0f = _sc_msg0(h0f, ex0, src, dst)
    out0 = out0f.reshape(H, N, D)

    rec0 = _recip(dp0, dp1)
    alpha0 = _sc_alpha(ex0, rec0, dst)

    h1, asad1 = _dense1(out0, dp0, dp1, W1, A1pad)
    ex1, dq0, dq1 = _sc_logits(asad1, src, dst)
    p0, p1 = _sc_msg1(h1, ex1, src, dst)

    out = _merge(p0, p1, dq0, dq1)
    anorm = _amean(alpha0.reshape(E, HP))
    return out.reshape(1, N, D), anorm.reshape(E)
